# Initial kernel scaffold; baseline (speedup 1.0000x reference)
import jax
import jax.numpy as jnp
from jax.experimental import pallas as pl


def kernel(x, l_e, edge_index, edge_label, mlp_v, mlp_e, mlp_edge1, mlp_edge0, mlp_aggr, gc_w, gc_b):
    raise NotImplementedError("write your pallas kernel here")


# speedup vs baseline: 2.1033x; 2.1033x over previous
import functools

import jax
import jax.numpy as jnp
from jax import lax
from jax.experimental import pallas as pl
from jax.experimental.pallas import tpu as pltpu
from jax.experimental.pallas import tpu_sc as plsc

F32 = jnp.float32
_NC, _NS = 2, 16
_NW = _NC * _NS
_TR = 800


def _dot(a, b):
    return jnp.dot(a, b, preferred_element_type=F32)


def _mlp3_body(x_ref, w1, b1, w2, b2, w3, b3, o_ref):
    h = jnp.maximum(_dot(x_ref[...], w1[...]) + b1[...], 0.0)
    h = jnp.maximum(_dot(h, w2[...]) + b2[...], 0.0)
    o_ref[...] = _dot(h, w3[...]) + b3[...]


def _tc_mlp3(x, p, bn):
    n, din = x.shape
    dout = p[4].shape[1]
    args = [x, p[0], p[1].reshape(1, -1), p[2], p[3].reshape(1, -1),
            p[4], p[5].reshape(1, -1)]
    in_specs = [pl.BlockSpec((bn, din), lambda i: (i, 0))]
    for a in args[1:]:
        in_specs.append(pl.BlockSpec(a.shape, lambda i: (0, 0)))
    return pl.pallas_call(
        _mlp3_body, grid=(n // bn,), in_specs=in_specs,
        out_specs=pl.BlockSpec((bn, dout), lambda i: (i, 0)),
        out_shape=jax.ShapeDtypeStruct((n, dout), F32),
    )(*args)


def _edge_embed_body(le_ref, mk_ref, w1, b1, w2, b2, w3, b3,
                     wc1, bc1, wc0, bc0, c_ref):
    h = jnp.maximum(le_ref[...] * w1[...] + b1[...], 0.0)
    h = jnp.maximum(_dot(h, w2[...]) + b2[...], 0.0)
    h = _dot(h, w3[...]) + b3[...]
    c1 = _dot(h, wc1[...]) + bc1[...]
    c0 = _dot(h, wc0[...]) + bc0[...]
    c_ref[...] = c0 + mk_ref[...] * (c1 - c0)


def _tc_edge_embed(l_e, mask_f, mlp_e, mlp_edge1, mlp_edge0, eb):
    e = l_e.shape[0]
    emb = mlp_e[4].shape[1]
    args = [l_e, mask_f,
            mlp_e[0], mlp_e[1].reshape(1, -1),
            mlp_e[2], mlp_e[3].reshape(1, -1),
            mlp_e[4], mlp_e[5].reshape(1, -1),
            mlp_edge1[0][2 * emb:], mlp_edge1[1].reshape(1, -1),
            mlp_edge0[0][2 * emb:], mlp_edge0[1].reshape(1, -1)]
    h1 = mlp_edge1[0].shape[1]
    in_specs = [pl.BlockSpec((eb, 1), lambda i: (i, 0)),
                pl.BlockSpec((eb, 1), lambda i: (i, 0))]
    for a in args[2:]:
        in_specs.append(pl.BlockSpec(a.shape, lambda i: (0, 0)))
    return pl.pallas_call(
        _edge_embed_body, grid=(e // eb,), in_specs=in_specs,
        out_specs=pl.BlockSpec((eb, h1), lambda i: (i, 0)),
        out_shape=jax.ShapeDtypeStruct((e, h1), F32),
    )(*args)


def _tables_body(h_ref, wa_ref, wb_ref, a_ref, b_ref):
    hh = h_ref[...]
    a_ref[...] = _dot(hh, wa_ref[0])
    b_ref[...] = _dot(hh, wb_ref[0])


def _tc_tables(h, wa_s, wb_s, bn):
    n, k = h.shape
    m = wa_s.shape[2]
    nb = n // bn
    return pl.pallas_call(
        _tables_body, grid=(2, nb),
        in_specs=[pl.BlockSpec((bn, k), lambda b, i: (i, 0)),
                  pl.BlockSpec((1, k, m), lambda b, i: (b, 0, 0)),
                  pl.BlockSpec((1, k, m), lambda b, i: (b, 0, 0))],
        out_specs=[pl.BlockSpec((bn, m), lambda b, i: (b * nb + i, 0)),
                   pl.BlockSpec((bn, m), lambda b, i: (b * nb + i, 0))],
        out_shape=[jax.ShapeDtypeStruct((2 * n, m), F32)] * 2,
    )(h, wa_s, wb_s)


def _edge_mlp_body(ga_ref, gb_ref, c_ref, mk_ref,
                   w21, b21, w31, b31, w20, b20, w30, b30, m_ref):
    z = jnp.maximum(ga_ref[...] + gb_ref[...] + c_ref[...], 0.0)
    u1 = jnp.maximum(_dot(z, w21[...]) + b21[...], 0.0)
    m1 = _dot(u1, w31[...]) + b31[...]
    u0 = jnp.maximum(_dot(z, w20[...]) + b20[...], 0.0)
    m0 = _dot(u0, w30[...]) + b30[...]
    m_ref[...] = m0 + mk_ref[...] * (m1 - m0)


def _tc_edge_mlp(ga, gb, c_sel, mask_f, mlp_edge1, mlp_edge0, eb):
    e, h1 = ga.shape
    emb = mlp_edge1[4].shape[1]
    args = [ga, gb, c_sel, mask_f,
            mlp_edge1[2], mlp_edge1[3].reshape(1, -1),
            mlp_edge1[4], mlp_edge1[5].reshape(1, -1),
            mlp_edge0[2], mlp_edge0[3].reshape(1, -1),
            mlp_edge0[4], mlp_edge0[5].reshape(1, -1)]
    in_specs = [pl.BlockSpec((eb, h1), lambda i: (i, 0)),
                pl.BlockSpec((eb, h1), lambda i: (i, 0)),
                pl.BlockSpec((eb, h1), lambda i: (i, 0)),
                pl.BlockSpec((eb, 1), lambda i: (i, 0))]
    for a in args[4:]:
        in_specs.append(pl.BlockSpec(a.shape, lambda i: (0, 0)))
    return pl.pallas_call(
        _edge_mlp_body, grid=(e // eb,), in_specs=in_specs,
        out_specs=pl.BlockSpec((eb, emb), lambda i: (i, 0)),
        out_shape=jax.ShapeDtypeStruct((e, emb), F32),
    )(*args)


def _aggr_body(h_ref, s1_ref, s0_ref, c1_ref, c0_ref,
               wh, w1, w0, b1, w2, b2, w3, b3, o_ref):
    hh = h_ref[...]
    sc1 = 1.0 / jnp.maximum(c1_ref[0][:, :1], 1.0)
    sc0 = 1.0 / jnp.maximum(c0_ref[0][:, :1], 1.0)
    u = _dot(hh, wh[...]) + b1[...]
    u += _dot(s1_ref[0] * sc1, w1[...])
    u += _dot(s0_ref[0] * sc0, w0[...])
    u = jnp.maximum(u, 0.0)
    u = jnp.maximum(_dot(u, w2[...]) + b2[...], 0.0)
    o_ref[...] = _dot(u, w3[...]) + b3[...] + hh


def _tc_aggr(h, s_mail, counts, mlp_aggr, bn):
    n, emb = h.shape
    nb = n // bn
    w = mlp_aggr[0]
    args = [h, s_mail, s_mail, counts, counts,
            w[:emb], w[emb:2 * emb], w[2 * emb:],
            mlp_aggr[1].reshape(1, -1),
            mlp_aggr[2], mlp_aggr[3].reshape(1, -1),
            mlp_aggr[4], mlp_aggr[5].reshape(1, -1)]
    in_specs = [pl.BlockSpec((bn, emb), lambda i: (i, 0)),
                pl.BlockSpec((1, bn, emb), lambda i: (0, i, 0)),
                pl.BlockSpec((1, bn, emb), lambda i: (1, i, 0)),
                pl.BlockSpec((1, bn, emb), lambda i: (0, i, 0)),
                pl.BlockSpec((1, bn, emb), lambda i: (1, i, 0))]
    for a in args[5:]:
        in_specs.append(pl.BlockSpec(a.shape, lambda i: (0, 0)))
    return pl.pallas_call(
        _aggr_body, grid=(nb,), in_specs=in_specs,
        out_specs=pl.BlockSpec((bn, emb), lambda i: (i, 0)),
        out_shape=jax.ShapeDtypeStruct((n, emb), F32),
    )(*args)


def _support_body(h_ref, w_ref, o_ref):
    o_ref[...] = _dot(h_ref[...], w_ref[0])


def _tc_support(h, gc_w, bn):
    n, emb = h.shape
    go = gc_w.shape[1]
    d = go // 2
    nb = n // bn
    w_s = jnp.stack([gc_w[:, :d], gc_w[:, d:]])
    return pl.pallas_call(
        _support_body, grid=(2, nb),
        in_specs=[pl.BlockSpec((bn, emb), lambda b, i: (i, 0)),
                  pl.BlockSpec((1, emb, d), lambda b, i: (b, 0, 0))],
        out_specs=pl.BlockSpec((bn, d), lambda b, i: (b * nb + i, 0)),
        out_shape=jax.ShapeDtypeStruct((2 * n, d), F32),
    )(h, w_s)


def _colmax_body(olo_ref, ohi_ref, blo, bhi, mlo_ref, mhi_ref):
    i = pl.program_id(0)

    @pl.when(i == 0)
    def _():
        mlo_ref[...] = jnp.full_like(mlo_ref[...], -jnp.inf)
        mhi_ref[...] = jnp.full_like(mhi_ref[...], -jnp.inf)

    mlo_ref[...] = jnp.maximum(
        mlo_ref[...], jnp.max(olo_ref[0] + blo[...], axis=0, keepdims=True))
    mhi_ref[...] = jnp.maximum(
        mhi_ref[...], jnp.max(ohi_ref[0] + bhi[...], axis=0, keepdims=True))


def _tc_colmax(o_acc, gc_b, bn):
    n = o_acc.shape[1]
    d = o_acc.shape[2]
    return pl.pallas_call(
        _colmax_body, grid=(n // bn,),
        in_specs=[pl.BlockSpec((1, bn, d), lambda i: (0, i, 0)),
                  pl.BlockSpec((1, bn, d), lambda i: (1, i, 0)),
                  pl.BlockSpec((1, d), lambda i: (0, 0)),
                  pl.BlockSpec((1, d), lambda i: (0, 0))],
        out_specs=[pl.BlockSpec((1, d), lambda i: (0, 0)),
                   pl.BlockSpec((1, d), lambda i: (0, 0))],
        out_shape=[jax.ShapeDtypeStruct((1, d), F32)] * 2,
    )(o_acc, o_acc, gc_b[:d].reshape(1, -1), gc_b[d:].reshape(1, -1))


_MESH = dict(core_axis_name="c", subcore_axis_name="s")


def _fill_const(buf, rows, cols, vec16):
    per_row = cols // 16

    def st(i, carry):
        buf[i // per_row, pl.ds((i % per_row) * 16, 16)] = vec16
        return carry

    lax.fori_loop(0, rows * per_row, st, 0)


def _sc_gather2(a_cat, b_cat, idx_a, idx_b):
    e = idx_a.shape[0]
    d = a_cat.shape[1]
    per_w = e // _NW
    ck = 40
    steps = per_w // ck

    @functools.partial(
        pl.kernel, mesh=plsc.VectorSubcoreMesh(**_MESH),
        out_type=[jax.ShapeDtypeStruct((e, d), F32)] * 2,
        scratch_types=[pltpu.VMEM((ck,), jnp.int32),
                       pltpu.VMEM((ck,), jnp.int32),
                       pltpu.VMEM((ck, d), F32),
                       pltpu.VMEM((ck, d), F32),
                       pltpu.SemaphoreType.DMA,
                       pltpu.SemaphoreType.DMA],
    )
    def k(a_hbm, b_hbm, ia_hbm, ib_hbm, ga_hbm, gb_hbm,
          ia_v, ib_v, bufa, bufb, sema, semb):
        wid = lax.axis_index("s") * _NC + lax.axis_index("c")
        w0 = wid * per_w

        def body(j, carry):
            base = w0 + j * ck
            pltpu.sync_copy(ia_hbm.at[pl.ds(base, ck)], ia_v)
            pltpu.sync_copy(ib_hbm.at[pl.ds(base, ck)], ib_v)
            da = pltpu.async_copy(a_hbm.at[ia_v], bufa, sema)
            db = pltpu.async_copy(b_hbm.at[ib_v], bufb, semb)
            da.wait()
            db.wait()
            pltpu.sync_copy(bufa, ga_hbm.at[pl.ds(base, ck)])
            pltpu.sync_copy(bufb, gb_hbm.at[pl.ds(base, ck)])
            return carry

        lax.fori_loop(0, steps, body, 0)

    return k(a_cat, b_cat, idx_a, idx_b)


def _sc_scatter_rowsplit(data, idx_cat, n):
    e, d = data.shape
    per_t = e // _NS
    ck = 80
    steps = per_t // ck
    zr = 200
    racc = n + _TR
    n_z = racc // zr
    zsteps = -(-n_z // _NS)
    n_ch = n // zr
    wsteps = -(-n_ch // _NS)

    @functools.partial(
        pl.kernel, mesh=plsc.VectorSubcoreMesh(**_MESH),
        out_type=jax.ShapeDtypeStruct((2, n, d), F32),
        scratch_types=[pltpu.VMEM((ck,), jnp.int32),
                       pltpu.VMEM((ck, d), F32),
                       pltpu.VMEM((zr, d), F32),
                       pltpu.VMEM_SHARED((racc, d), F32)],
    )
    def k(m_hbm, idx_hbm, out_hbm, idx_v, dbuf, zbuf, acc):
        c = lax.axis_index("c")
        s = lax.axis_index("s")
        _fill_const(zbuf, zr, d, jnp.zeros((16,), F32))

        def zc(i, carry):
            cid = s + i * _NS

            @pl.when(cid < n_z)
            def _():
                pltpu.sync_copy(zbuf, acc.at[pl.ds(cid * zr, zr)])

            return carry

        lax.fori_loop(0, zsteps, zc, 0)
        plsc.subcore_barrier()

        def body(j, carry):
            base = s * per_t + j * ck
            pltpu.sync_copy(idx_hbm.at[pl.ds(c * e + base, ck)], idx_v)
            pltpu.sync_copy(m_hbm.at[pl.ds(base, ck)], dbuf)
            pltpu.sync_copy(dbuf, acc.at[idx_v], add=True)
            return carry

        lax.fori_loop(0, steps, body, 0)
        plsc.subcore_barrier()

        def wb(i, carry):
            cid = s + i * _NS

            @pl.when(cid < n_ch)
            def _():
                r0 = cid * zr
                pltpu.sync_copy(acc.at[pl.ds(r0, zr)], zbuf)

                @pl.when(c == 0)
                def _():
                    pltpu.sync_copy(zbuf, out_hbm.at[0, pl.ds(r0, zr)])

                @pl.when(c == 1)
                def _():
                    pltpu.sync_copy(zbuf, out_hbm.at[1, pl.ds(r0, zr)])

            return carry

        lax.fori_loop(0, wsteps, wb, 0)

    return k(data, idx_cat)


def _sc_counts(idx_cat, e, n):
    d = 128
    per_t = e // _NS
    ck = 80
    steps = per_t // ck
    zr = 200
    racc = n + _TR
    n_z = racc // zr
    zsteps = -(-n_z // _NS)
    n_ch = n // zr
    wsteps = -(-n_ch // _NS)

    @functools.partial(
        pl.kernel, mesh=plsc.VectorSubcoreMesh(**_MESH),
        out_type=jax.ShapeDtypeStruct((2, n, d), F32),
        scratch_types=[pltpu.VMEM((ck,), jnp.int32),
                       pltpu.VMEM((ck, d), F32),
                       pltpu.VMEM((zr, d), F32),
                       pltpu.VMEM_SHARED((racc, d), F32)],
    )
    def k(idx_hbm, out_hbm, idx_v, ones_v, zbuf, acc):
        c = lax.axis_index("c")
        s = lax.axis_index("s")
        _fill_const(ones_v, ck, d, jnp.ones((16,), F32))
        _fill_const(zbuf, zr, d, jnp.zeros((16,), F32))

        def zc(i, carry):
            cid = s + i * _NS

            @pl.when(cid < n_z)
            def _():
                pltpu.sync_copy(zbuf, acc.at[pl.ds(cid * zr, zr)])

            return carry

        lax.fori_loop(0, zsteps, zc, 0)
        plsc.subcore_barrier()

        def body(j, carry):
            base = c * e + s * per_t + j * ck
            pltpu.sync_copy(idx_hbm.at[pl.ds(base, ck)], idx_v)
            pltpu.sync_copy(ones_v, acc.at[idx_v], add=True)
            return carry

        lax.fori_loop(0, steps, body, 0)
        plsc.subcore_barrier()

        def wb(i, carry):
            cid = s + i * _NS

            @pl.when(cid < n_ch)
            def _():
                r0 = cid * zr
                pltpu.sync_copy(acc.at[pl.ds(r0, zr)], zbuf)

                @pl.when(c == 0)
                def _():
                    pltpu.sync_copy(zbuf, out_hbm.at[0, pl.ds(r0, zr)])

                @pl.when(c == 1)
                def _():
                    pltpu.sync_copy(zbuf, out_hbm.at[1, pl.ds(r0, zr)])

            return carry

        lax.fori_loop(0, wsteps, wb, 0)

    return k(idx_cat)


def _sc_spmm(sup_cat, src2, dst, n):
    e = dst.shape[0]
    d = sup_cat.shape[1]
    per_t = e // _NS
    ck = 80
    steps = per_t // ck
    zr = 200
    n_ch = n // zr
    wsteps = -(-n_ch // _NS)

    @functools.partial(
        pl.kernel, mesh=plsc.VectorSubcoreMesh(**_MESH),
        out_type=jax.ShapeDtypeStruct((2, n, d), F32),
        scratch_types=[pltpu.VMEM((ck,), jnp.int32),
                       pltpu.VMEM((ck,), jnp.int32),
                       pltpu.VMEM((ck, d), F32),
                       pltpu.VMEM((zr, d), F32),
                       pltpu.VMEM_SHARED((n, d), F32),
                       pltpu.SemaphoreType.DMA],
    )
    def k(sup_hbm, src_hbm, dst_hbm, out_hbm, is_v, id_v, gbuf, zbuf, acc, sem):
        c = lax.axis_index("c")
        s = lax.axis_index("s")
        _fill_const(zbuf, zr, d, jnp.zeros((16,), F32))

        def zc(i, carry):
            cid = s + i * _NS

            @pl.when(cid < n_ch)
            def _():
                pltpu.sync_copy(zbuf, acc.at[pl.ds(cid * zr, zr)])

            return carry

        lax.fori_loop(0, wsteps, zc, 0)
        plsc.subcore_barrier()

        def body(j, carry):
            base = s * per_t + j * ck
            pltpu.sync_copy(src_hbm.at[pl.ds(c * e + base, ck)], is_v)
            pltpu.sync_copy(dst_hbm.at[pl.ds(base, ck)], id_v)
            pltpu.async_copy(sup_hbm.at[is_v], gbuf, sem).wait()
            pltpu.sync_copy(gbuf, acc.at[id_v], add=True)
            return carry

        lax.fori_loop(0, steps, body, 0)
        plsc.subcore_barrier()

        def wb(i, carry):
            cid = s + i * _NS

            @pl.when(cid < n_ch)
            def _():
                r0 = cid * zr
                pltpu.sync_copy(acc.at[pl.ds(r0, zr)], zbuf)

                @pl.when(c == 0)
                def _():
                    pltpu.sync_copy(zbuf, out_hbm.at[0, pl.ds(r0, zr)])

                @pl.when(c == 1)
                def _():
                    pltpu.sync_copy(zbuf, out_hbm.at[1, pl.ds(r0, zr)])

            return carry

        lax.fori_loop(0, wsteps, wb, 0)

    return k(sup_cat, src2, dst)


def kernel(x, l_e, edge_index, edge_label, mlp_v, mlp_e, mlp_edge1,
           mlp_edge0, mlp_aggr, gc_w, gc_b):
    n, emb = x.shape
    e = l_e.shape[0]
    bn = 2000
    eb = 1000
    src = edge_index[0].astype(jnp.int32)
    dst = edge_index[1].astype(jnp.int32)
    mask_f = (edge_label == 1).astype(F32)[:, None]
    off = jnp.where(edge_label == 1, 0, n).astype(jnp.int32)
    src_adj = src + off
    dst_adj = dst + off
    trash = (n + dst % _TR).astype(jnp.int32)
    idx_mail = jnp.concatenate([
        jnp.where(edge_label == 1, dst, trash),
        jnp.where(edge_label == 0, dst, trash)]).astype(jnp.int32)
    src2 = jnp.concatenate([src, src + n]).astype(jnp.int32)

    h = _tc_mlp3(x, mlp_v, bn)
    c_sel = _tc_edge_embed(l_e, mask_f, mlp_e, mlp_edge1, mlp_edge0, eb)
    counts = _sc_counts(idx_mail, e, n)

    wa_s = jnp.stack([mlp_edge1[0][:emb], mlp_edge0[0][:emb]])
    wb_s = jnp.stack([mlp_edge1[0][emb:2 * emb], mlp_edge0[0][emb:2 * emb]])

    for _ in range(2):
        a_cat, b_cat = _tc_tables(h, wa_s, wb_s, bn)
        ga, gb = _sc_gather2(a_cat, b_cat, src_adj, dst_adj)
        m = _tc_edge_mlp(ga, gb, c_sel, mask_f, mlp_edge1, mlp_edge0, eb)
        s_mail = _sc_scatter_rowsplit(m, idx_mail, n)
        h = _tc_aggr(h, s_mail, counts, mlp_aggr, bn)

    sup_cat = _tc_support(h, gc_w, bn)
    o_acc = _sc_spmm(sup_cat, src2, dst, n)
    mlo, mhi = _tc_colmax(o_acc, gc_b, bn)
    return jnp.concatenate([mlo[0], mhi[0]], axis=0)


# speedup vs baseline: 2.5496x; 1.2122x over previous
import functools

import jax
import jax.numpy as jnp
from jax import lax
from jax.experimental import pallas as pl
from jax.experimental.pallas import tpu as pltpu
from jax.experimental.pallas import tpu_sc as plsc

F32 = jnp.float32
_NC, _NS = 2, 16
_NW = _NC * _NS
_TR = 800


def _dot(a, b):
    return jnp.dot(a, b, preferred_element_type=F32)


def _mlp3_body(x_ref, w1, b1, w2, b2, w3, b3, o_ref):
    h = jnp.maximum(_dot(x_ref[...], w1[...]) + b1[...], 0.0)
    h = jnp.maximum(_dot(h, w2[...]) + b2[...], 0.0)
    o_ref[...] = _dot(h, w3[...]) + b3[...]


def _tc_mlp3(x, p, bn):
    n, din = x.shape
    dout = p[4].shape[1]
    args = [x, p[0], p[1].reshape(1, -1), p[2], p[3].reshape(1, -1),
            p[4], p[5].reshape(1, -1)]
    in_specs = [pl.BlockSpec((bn, din), lambda i: (i, 0))]
    for a in args[1:]:
        in_specs.append(pl.BlockSpec(a.shape, lambda i: (0, 0)))
    return pl.pallas_call(
        _mlp3_body, grid=(n // bn,), in_specs=in_specs,
        out_specs=pl.BlockSpec((bn, dout), lambda i: (i, 0)),
        out_shape=jax.ShapeDtypeStruct((n, dout), F32),
    )(*args)


def _edge_embed_body(le_ref, mk_ref, w1, b1, w2, b2, w3, b3,
                     wc1, bc1, wc0, bc0, c_ref):
    h = jnp.maximum(le_ref[...] * w1[...] + b1[...], 0.0)
    h = jnp.maximum(_dot(h, w2[...]) + b2[...], 0.0)
    h = _dot(h, w3[...]) + b3[...]
    c1 = _dot(h, wc1[...]) + bc1[...]
    c0 = _dot(h, wc0[...]) + bc0[...]
    c_ref[...] = c0 + mk_ref[...] * (c1 - c0)


def _tc_edge_embed(l_e, mask_f, mlp_e, mlp_edge1, mlp_edge0, eb):
    e = l_e.shape[0]
    emb = mlp_e[4].shape[1]
    args = [l_e, mask_f,
            mlp_e[0], mlp_e[1].reshape(1, -1),
            mlp_e[2], mlp_e[3].reshape(1, -1),
            mlp_e[4], mlp_e[5].reshape(1, -1),
            mlp_edge1[0][2 * emb:], mlp_edge1[1].reshape(1, -1),
            mlp_edge0[0][2 * emb:], mlp_edge0[1].reshape(1, -1)]
    h1 = mlp_edge1[0].shape[1]
    in_specs = [pl.BlockSpec((eb, 1), lambda i: (i, 0)),
                pl.BlockSpec((eb, 1), lambda i: (i, 0))]
    for a in args[2:]:
        in_specs.append(pl.BlockSpec(a.shape, lambda i: (0, 0)))
    return pl.pallas_call(
        _edge_embed_body, grid=(e // eb,), in_specs=in_specs,
        out_specs=pl.BlockSpec((eb, h1), lambda i: (i, 0)),
        out_shape=jax.ShapeDtypeStruct((e, h1), F32),
    )(*args)


def _tables_body(h_ref, wa_ref, wb_ref, a_ref, b_ref):
    hh = h_ref[...]
    a_ref[...] = _dot(hh, wa_ref[0])
    b_ref[...] = _dot(hh, wb_ref[0])


def _tc_tables(h, wa_s, wb_s, bn):
    n, k = h.shape
    m = wa_s.shape[2]
    nb = n // bn
    return pl.pallas_call(
        _tables_body, grid=(2, nb),
        in_specs=[pl.BlockSpec((bn, k), lambda b, i: (i, 0)),
                  pl.BlockSpec((1, k, m), lambda b, i: (b, 0, 0)),
                  pl.BlockSpec((1, k, m), lambda b, i: (b, 0, 0))],
        out_specs=[pl.BlockSpec((bn, m), lambda b, i: (b * nb + i, 0)),
                   pl.BlockSpec((bn, m), lambda b, i: (b * nb + i, 0))],
        out_shape=[jax.ShapeDtypeStruct((2 * n, m), F32)] * 2,
    )(h, wa_s, wb_s)


def _edge_mlp_body(ga_ref, gb_ref, c_ref, mk_ref,
                   w21, b21, w31, b31, w20, b20, w30, b30, m_ref):
    z = jnp.maximum(ga_ref[...] + gb_ref[...] + c_ref[...], 0.0)
    u1 = jnp.maximum(_dot(z, w21[...]) + b21[...], 0.0)
    m1 = _dot(u1, w31[...]) + b31[...]
    u0 = jnp.maximum(_dot(z, w20[...]) + b20[...], 0.0)
    m0 = _dot(u0, w30[...]) + b30[...]
    m_ref[...] = m0 + mk_ref[...] * (m1 - m0)


def _tc_edge_mlp(ga, gb, c_sel, mask_f, mlp_edge1, mlp_edge0, eb):
    e, h1 = ga.shape
    emb = mlp_edge1[4].shape[1]
    args = [ga, gb, c_sel, mask_f,
            mlp_edge1[2], mlp_edge1[3].reshape(1, -1),
            mlp_edge1[4], mlp_edge1[5].reshape(1, -1),
            mlp_edge0[2], mlp_edge0[3].reshape(1, -1),
            mlp_edge0[4], mlp_edge0[5].reshape(1, -1)]
    in_specs = [pl.BlockSpec((eb, h1), lambda i: (i, 0)),
                pl.BlockSpec((eb, h1), lambda i: (i, 0)),
                pl.BlockSpec((eb, h1), lambda i: (i, 0)),
                pl.BlockSpec((eb, 1), lambda i: (i, 0))]
    for a in args[4:]:
        in_specs.append(pl.BlockSpec(a.shape, lambda i: (0, 0)))
    return pl.pallas_call(
        _edge_mlp_body, grid=(e // eb,), in_specs=in_specs,
        out_specs=pl.BlockSpec((eb, emb), lambda i: (i, 0)),
        out_shape=jax.ShapeDtypeStruct((e, emb), F32),
    )(*args)


def _aggr_body(h_ref, s1_ref, s0_ref, c1_ref, c0_ref,
               wh, w1, w0, b1, w2, b2, w3, b3, o_ref):
    hh = h_ref[...]
    sc1 = 1.0 / jnp.maximum(c1_ref[0][:, :1], 1.0)
    sc0 = 1.0 / jnp.maximum(c0_ref[0][:, :1], 1.0)
    u = _dot(hh, wh[...]) + b1[...]
    u += _dot(s1_ref[0] * sc1, w1[...])
    u += _dot(s0_ref[0] * sc0, w0[...])
    u = jnp.maximum(u, 0.0)
    u = jnp.maximum(_dot(u, w2[...]) + b2[...], 0.0)
    o_ref[...] = _dot(u, w3[...]) + b3[...] + hh


def _tc_aggr(h, s_mail, counts, mlp_aggr, bn):
    n, emb = h.shape
    nb = n // bn
    w = mlp_aggr[0]
    args = [h, s_mail, s_mail, counts, counts,
            w[:emb], w[emb:2 * emb], w[2 * emb:],
            mlp_aggr[1].reshape(1, -1),
            mlp_aggr[2], mlp_aggr[3].reshape(1, -1),
            mlp_aggr[4], mlp_aggr[5].reshape(1, -1)]
    in_specs = [pl.BlockSpec((bn, emb), lambda i: (i, 0)),
                pl.BlockSpec((1, bn, emb), lambda i: (0, i, 0)),
                pl.BlockSpec((1, bn, emb), lambda i: (1, i, 0)),
                pl.BlockSpec((1, bn, emb), lambda i: (0, i, 0)),
                pl.BlockSpec((1, bn, emb), lambda i: (1, i, 0))]
    for a in args[5:]:
        in_specs.append(pl.BlockSpec(a.shape, lambda i: (0, 0)))
    return pl.pallas_call(
        _aggr_body, grid=(nb,), in_specs=in_specs,
        out_specs=pl.BlockSpec((bn, emb), lambda i: (i, 0)),
        out_shape=jax.ShapeDtypeStruct((n, emb), F32),
    )(*args)


def _support_body(h_ref, w_ref, o_ref):
    o_ref[...] = _dot(h_ref[...], w_ref[0])


def _tc_support(h, gc_w, bn):
    n, emb = h.shape
    go = gc_w.shape[1]
    d = go // 2
    nb = n // bn
    w_s = jnp.stack([gc_w[:, :d], gc_w[:, d:]])
    return pl.pallas_call(
        _support_body, grid=(2, nb),
        in_specs=[pl.BlockSpec((bn, emb), lambda b, i: (i, 0)),
                  pl.BlockSpec((1, emb, d), lambda b, i: (b, 0, 0))],
        out_specs=pl.BlockSpec((bn, d), lambda b, i: (b * nb + i, 0)),
        out_shape=jax.ShapeDtypeStruct((2 * n, d), F32),
    )(h, w_s)


def _colmax_body(olo_ref, ohi_ref, blo, bhi, mlo_ref, mhi_ref):
    i = pl.program_id(0)

    @pl.when(i == 0)
    def _():
        mlo_ref[...] = jnp.full_like(mlo_ref[...], -jnp.inf)
        mhi_ref[...] = jnp.full_like(mhi_ref[...], -jnp.inf)

    mlo_ref[...] = jnp.maximum(
        mlo_ref[...], jnp.max(olo_ref[0] + blo[...], axis=0, keepdims=True))
    mhi_ref[...] = jnp.maximum(
        mhi_ref[...], jnp.max(ohi_ref[0] + bhi[...], axis=0, keepdims=True))


def _tc_colmax(o_acc, gc_b, bn):
    n = o_acc.shape[1]
    d = o_acc.shape[2]
    return pl.pallas_call(
        _colmax_body, grid=(n // bn,),
        in_specs=[pl.BlockSpec((1, bn, d), lambda i: (0, i, 0)),
                  pl.BlockSpec((1, bn, d), lambda i: (1, i, 0)),
                  pl.BlockSpec((1, d), lambda i: (0, 0)),
                  pl.BlockSpec((1, d), lambda i: (0, 0))],
        out_specs=[pl.BlockSpec((1, d), lambda i: (0, 0)),
                   pl.BlockSpec((1, d), lambda i: (0, 0))],
        out_shape=[jax.ShapeDtypeStruct((1, d), F32)] * 2,
    )(o_acc, o_acc, gc_b[:d].reshape(1, -1), gc_b[d:].reshape(1, -1))


_MESH = dict(core_axis_name="c", subcore_axis_name="s")


def _fill_const(buf, rows, cols, vec16):
    per_row = cols // 16

    def st(i, carry):
        buf[i // per_row, pl.ds((i % per_row) * 16, 16)] = vec16
        return carry

    lax.fori_loop(0, rows * per_row, st, 0)


def _sc_gather2(a_cat, b_cat, idx_a, idx_b):
    e = idx_a.shape[0]
    d = a_cat.shape[1]
    per_w = e // _NW
    ck = 224
    steps = -(-per_w // ck)
    last = per_w - ck

    @functools.partial(
        pl.kernel, mesh=plsc.VectorSubcoreMesh(**_MESH),
        out_type=[jax.ShapeDtypeStruct((e, d), F32)] * 2,
        scratch_types=[pltpu.VMEM((ck,), jnp.int32),
                       pltpu.VMEM((ck,), jnp.int32),
                       pltpu.VMEM((ck, d), F32),
                       pltpu.VMEM((ck, d), F32),
                       pltpu.SemaphoreType.DMA,
                       pltpu.SemaphoreType.DMA],
    )
    def k(a_hbm, b_hbm, ia_hbm, ib_hbm, ga_hbm, gb_hbm,
          ia_v, ib_v, bufa, bufb, sema, semb):
        wid = lax.axis_index("s") * _NC + lax.axis_index("c")
        w0 = wid * per_w

        def body(j, carry):
            base = w0 + jnp.minimum(j * ck, last)
            pltpu.sync_copy(ia_hbm.at[pl.ds(base, ck)], ia_v)
            pltpu.sync_copy(ib_hbm.at[pl.ds(base, ck)], ib_v)
            da = pltpu.async_copy(a_hbm.at[ia_v], bufa, sema)
            db = pltpu.async_copy(b_hbm.at[ib_v], bufb, semb)
            da.wait()
            db.wait()
            pltpu.sync_copy(bufa, ga_hbm.at[pl.ds(base, ck)])
            pltpu.sync_copy(bufb, gb_hbm.at[pl.ds(base, ck)])
            return carry

        lax.fori_loop(0, steps, body, 0)

    return k(a_cat, b_cat, idx_a, idx_b)


def _sc_scatter_rowsplit(data, idx_cat, n):
    e, d = data.shape
    per_t = e // _NS
    ck = 200
    steps = per_t // ck
    zr = ck
    racc = n + _TR
    n_z = racc // zr
    zsteps = -(-n_z // _NS)
    n_ch = n // zr
    wsteps = -(-n_ch // _NS)

    @functools.partial(
        pl.kernel, mesh=plsc.VectorSubcoreMesh(**_MESH),
        out_type=jax.ShapeDtypeStruct((2, n, d), F32),
        scratch_types=[pltpu.VMEM((ck,), jnp.int32),
                       pltpu.VMEM((ck, d), F32),
                       pltpu.VMEM_SHARED((racc, d), F32)],
    )
    def k(m_hbm, idx_hbm, out_hbm, idx_v, dbuf, acc):
        c = lax.axis_index("c")
        s = lax.axis_index("s")
        _fill_const(dbuf, zr, d, jnp.zeros((16,), F32))

        def zc(i, carry):
            cid = s + i * _NS

            @pl.when(cid < n_z)
            def _():
                pltpu.sync_copy(dbuf, acc.at[pl.ds(cid * zr, zr)])

            return carry

        lax.fori_loop(0, zsteps, zc, 0)
        plsc.subcore_barrier()

        def body(j, carry):
            base = s * per_t + j * ck
            pltpu.sync_copy(idx_hbm.at[pl.ds(c * e + base, ck)], idx_v)
            pltpu.sync_copy(m_hbm.at[pl.ds(base, ck)], dbuf)
            pltpu.sync_copy(dbuf, acc.at[idx_v], add=True)
            return carry

        lax.fori_loop(0, steps, body, 0)
        plsc.subcore_barrier()

        def wb(i, carry):
            cid = s + i * _NS

            @pl.when(cid < n_ch)
            def _():
                r0 = cid * zr
                pltpu.sync_copy(acc.at[pl.ds(r0, zr)], dbuf)

                @pl.when(c == 0)
                def _():
                    pltpu.sync_copy(dbuf, out_hbm.at[0, pl.ds(r0, zr)])

                @pl.when(c == 1)
                def _():
                    pltpu.sync_copy(dbuf, out_hbm.at[1, pl.ds(r0, zr)])

            return carry

        lax.fori_loop(0, wsteps, wb, 0)

    return k(data, idx_cat)


def _sc_counts(idx_cat, e, n):
    d = 128
    per_t = e // _NS
    ck = 200
    steps = per_t // ck
    zr = ck
    racc = n + _TR
    n_z = racc // zr
    zsteps = -(-n_z // _NS)
    n_ch = n // zr
    wsteps = -(-n_ch // _NS)

    @functools.partial(
        pl.kernel, mesh=plsc.VectorSubcoreMesh(**_MESH),
        out_type=jax.ShapeDtypeStruct((2, n, d), F32),
        scratch_types=[pltpu.VMEM((ck,), jnp.int32),
                       pltpu.VMEM((ck, d), F32),
                       pltpu.VMEM_SHARED((racc, d), F32)],
    )
    def k(idx_hbm, out_hbm, idx_v, ones_v, acc):
        c = lax.axis_index("c")
        s = lax.axis_index("s")
        _fill_const(ones_v, zr, d, jnp.zeros((16,), F32))

        def zc(i, carry):
            cid = s + i * _NS

            @pl.when(cid < n_z)
            def _():
                pltpu.sync_copy(ones_v, acc.at[pl.ds(cid * zr, zr)])

            return carry

        lax.fori_loop(0, zsteps, zc, 0)
        plsc.subcore_barrier()
        _fill_const(ones_v, ck, d, jnp.ones((16,), F32))

        def body(j, carry):
            base = c * e + s * per_t + j * ck
            pltpu.sync_copy(idx_hbm.at[pl.ds(base, ck)], idx_v)
            pltpu.sync_copy(ones_v, acc.at[idx_v], add=True)
            return carry

        lax.fori_loop(0, steps, body, 0)
        plsc.subcore_barrier()

        def wb(i, carry):
            cid = s + i * _NS

            @pl.when(cid < n_ch)
            def _():
                r0 = cid * zr
                pltpu.sync_copy(acc.at[pl.ds(r0, zr)], ones_v)

                @pl.when(c == 0)
                def _():
                    pltpu.sync_copy(ones_v, out_hbm.at[0, pl.ds(r0, zr)])

                @pl.when(c == 1)
                def _():
                    pltpu.sync_copy(ones_v, out_hbm.at[1, pl.ds(r0, zr)])

            return carry

        lax.fori_loop(0, wsteps, wb, 0)

    return k(idx_cat)


def _sc_spmm(sup_cat, src2, dst, n):
    e = dst.shape[0]
    d = sup_cat.shape[1]
    per_t = e // _NS
    ck = 200
    steps = per_t // ck
    zr = ck
    n_ch = n // zr
    wsteps = -(-n_ch // _NS)

    @functools.partial(
        pl.kernel, mesh=plsc.VectorSubcoreMesh(**_MESH),
        out_type=jax.ShapeDtypeStruct((2, n, d), F32),
        scratch_types=[pltpu.VMEM((ck,), jnp.int32),
                       pltpu.VMEM((ck,), jnp.int32),
                       pltpu.VMEM((ck, d), F32),
                       pltpu.VMEM_SHARED((n, d), F32),
                       pltpu.SemaphoreType.DMA],
    )
    def k(sup_hbm, src_hbm, dst_hbm, out_hbm, is_v, id_v, gbuf, acc, sem):
        c = lax.axis_index("c")
        s = lax.axis_index("s")
        _fill_const(gbuf, zr, d, jnp.zeros((16,), F32))

        def zc(i, carry):
            cid = s + i * _NS

            @pl.when(cid < n_ch)
            def _():
                pltpu.sync_copy(gbuf, acc.at[pl.ds(cid * zr, zr)])

            return carry

        lax.fori_loop(0, wsteps, zc, 0)
        plsc.subcore_barrier()

        def body(j, carry):
            base = s * per_t + j * ck
            pltpu.sync_copy(src_hbm.at[pl.ds(c * e + base, ck)], is_v)
            pltpu.sync_copy(dst_hbm.at[pl.ds(base, ck)], id_v)
            pltpu.async_copy(sup_hbm.at[is_v], gbuf, sem).wait()
            pltpu.sync_copy(gbuf, acc.at[id_v], add=True)
            return carry

        lax.fori_loop(0, steps, body, 0)
        plsc.subcore_barrier()

        def wb(i, carry):
            cid = s + i * _NS

            @pl.when(cid < n_ch)
            def _():
                r0 = cid * zr
                pltpu.sync_copy(acc.at[pl.ds(r0, zr)], gbuf)

                @pl.when(c == 0)
                def _():
                    pltpu.sync_copy(gbuf, out_hbm.at[0, pl.ds(r0, zr)])

                @pl.when(c == 1)
                def _():
                    pltpu.sync_copy(gbuf, out_hbm.at[1, pl.ds(r0, zr)])

            return carry

        lax.fori_loop(0, wsteps, wb, 0)

    return k(sup_cat, src2, dst)


def kernel(x, l_e, edge_index, edge_label, mlp_v, mlp_e, mlp_edge1,
           mlp_edge0, mlp_aggr, gc_w, gc_b):
    n, emb = x.shape
    e = l_e.shape[0]
    bn = 2000
    eb = 1000
    src = edge_index[0].astype(jnp.int32)
    dst = edge_index[1].astype(jnp.int32)
    mask_f = (edge_label == 1).astype(F32)[:, None]
    off = jnp.where(edge_label == 1, 0, n).astype(jnp.int32)
    src_adj = src + off
    dst_adj = dst + off
    trash = (n + dst % _TR).astype(jnp.int32)
    idx_mail = jnp.concatenate([
        jnp.where(edge_label == 1, dst, trash),
        jnp.where(edge_label == 0, dst, trash)]).astype(jnp.int32)
    src2 = jnp.concatenate([src, src + n]).astype(jnp.int32)

    h = _tc_mlp3(x, mlp_v, bn)
    c_sel = _tc_edge_embed(l_e, mask_f, mlp_e, mlp_edge1, mlp_edge0, eb)
    counts = _sc_counts(idx_mail, e, n)

    wa_s = jnp.stack([mlp_edge1[0][:emb], mlp_edge0[0][:emb]])
    wb_s = jnp.stack([mlp_edge1[0][emb:2 * emb], mlp_edge0[0][emb:2 * emb]])

    for _ in range(2):
        a_cat, b_cat = _tc_tables(h, wa_s, wb_s, bn)
        ga, gb = _sc_gather2(a_cat, b_cat, src_adj, dst_adj)
        m = _tc_edge_mlp(ga, gb, c_sel, mask_f, mlp_edge1, mlp_edge0, eb)
        s_mail = _sc_scatter_rowsplit(m, idx_mail, n)
        h = _tc_aggr(h, s_mail, counts, mlp_aggr, bn)

    sup_cat = _tc_support(h, gc_w, bn)
    o_acc = _sc_spmm(sup_cat, src2, dst, n)
    mlo, mhi = _tc_colmax(o_acc, gc_b, bn)
    return jnp.concatenate([mlo[0], mhi[0]], axis=0)


# speedup vs baseline: 2.5708x; 1.0083x over previous
import functools

import jax
import jax.numpy as jnp
from jax import lax
from jax.experimental import pallas as pl
from jax.experimental.pallas import tpu as pltpu
from jax.experimental.pallas import tpu_sc as plsc

F32 = jnp.float32
_NC, _NS = 2, 16
_NW = _NC * _NS
_TR = 800


def _dot(a, b):
    return jnp.dot(a.astype(jnp.bfloat16), b.astype(jnp.bfloat16),
                   preferred_element_type=F32)


def _mlp3_body(x_ref, w1, b1, w2, b2, w3, b3, o_ref):
    h = jnp.maximum(_dot(x_ref[...], w1[...]) + b1[...], 0.0)
    h = jnp.maximum(_dot(h, w2[...]) + b2[...], 0.0)
    o_ref[...] = _dot(h, w3[...]) + b3[...]


def _tc_mlp3(x, p, bn):
    n, din = x.shape
    dout = p[4].shape[1]
    args = [x, p[0], p[1].reshape(1, -1), p[2], p[3].reshape(1, -1),
            p[4], p[5].reshape(1, -1)]
    in_specs = [pl.BlockSpec((bn, din), lambda i: (i, 0))]
    for a in args[1:]:
        in_specs.append(pl.BlockSpec(a.shape, lambda i: (0, 0)))
    return pl.pallas_call(
        _mlp3_body, grid=(n // bn,), in_specs=in_specs,
        out_specs=pl.BlockSpec((bn, dout), lambda i: (i, 0)),
        out_shape=jax.ShapeDtypeStruct((n, dout), F32),
    )(*args)


def _edge_embed_body(le_ref, mk_ref, w1, b1, w2, b2, w3, b3,
                     wc1, bc1, wc0, bc0, c_ref):
    h = jnp.maximum(le_ref[...] * w1[...] + b1[...], 0.0)
    h = jnp.maximum(_dot(h, w2[...]) + b2[...], 0.0)
    h = _dot(h, w3[...]) + b3[...]
    c1 = _dot(h, wc1[...]) + bc1[...]
    c0 = _dot(h, wc0[...]) + bc0[...]
    c_ref[...] = c0 + mk_ref[...] * (c1 - c0)


def _tc_edge_embed(l_e, mask_f, mlp_e, mlp_edge1, mlp_edge0, eb):
    e = l_e.shape[0]
    emb = mlp_e[4].shape[1]
    args = [l_e, mask_f,
            mlp_e[0], mlp_e[1].reshape(1, -1),
            mlp_e[2], mlp_e[3].reshape(1, -1),
            mlp_e[4], mlp_e[5].reshape(1, -1),
            mlp_edge1[0][2 * emb:], mlp_edge1[1].reshape(1, -1),
            mlp_edge0[0][2 * emb:], mlp_edge0[1].reshape(1, -1)]
    h1 = mlp_edge1[0].shape[1]
    in_specs = [pl.BlockSpec((eb, 1), lambda i: (i, 0)),
                pl.BlockSpec((eb, 1), lambda i: (i, 0))]
    for a in args[2:]:
        in_specs.append(pl.BlockSpec(a.shape, lambda i: (0, 0)))
    return pl.pallas_call(
        _edge_embed_body, grid=(e // eb,), in_specs=in_specs,
        out_specs=pl.BlockSpec((eb, h1), lambda i: (i, 0)),
        out_shape=jax.ShapeDtypeStruct((e, h1), F32),
    )(*args)


def _tables_body(h_ref, wa_ref, wb_ref, a_ref, b_ref):
    hh = h_ref[...]
    a_ref[...] = _dot(hh, wa_ref[0])
    b_ref[...] = _dot(hh, wb_ref[0])


def _tc_tables(h, wa_s, wb_s, bn):
    n, k = h.shape
    m = wa_s.shape[2]
    nb = n // bn
    return pl.pallas_call(
        _tables_body, grid=(2, nb),
        in_specs=[pl.BlockSpec((bn, k), lambda b, i: (i, 0)),
                  pl.BlockSpec((1, k, m), lambda b, i: (b, 0, 0)),
                  pl.BlockSpec((1, k, m), lambda b, i: (b, 0, 0))],
        out_specs=[pl.BlockSpec((bn, m), lambda b, i: (b * nb + i, 0)),
                   pl.BlockSpec((bn, m), lambda b, i: (b * nb + i, 0))],
        out_shape=[jax.ShapeDtypeStruct((2 * n, m), F32)] * 2,
    )(h, wa_s, wb_s)


def _edge_mlp_body(ga_ref, gb_ref, c_ref, mk_ref,
                   w21, b21, w31, b31, w20, b20, w30, b30, m_ref):
    z = jnp.maximum(ga_ref[...] + gb_ref[...] + c_ref[...], 0.0)
    u1 = jnp.maximum(_dot(z, w21[...]) + b21[...], 0.0)
    m1 = _dot(u1, w31[...]) + b31[...]
    u0 = jnp.maximum(_dot(z, w20[...]) + b20[...], 0.0)
    m0 = _dot(u0, w30[...]) + b30[...]
    m_ref[...] = m0 + mk_ref[...] * (m1 - m0)


def _tc_edge_mlp(ga, gb, c_sel, mask_f, mlp_edge1, mlp_edge0, eb):
    e, h1 = ga.shape
    emb = mlp_edge1[4].shape[1]
    args = [ga, gb, c_sel, mask_f,
            mlp_edge1[2], mlp_edge1[3].reshape(1, -1),
            mlp_edge1[4], mlp_edge1[5].reshape(1, -1),
            mlp_edge0[2], mlp_edge0[3].reshape(1, -1),
            mlp_edge0[4], mlp_edge0[5].reshape(1, -1)]
    in_specs = [pl.BlockSpec((eb, h1), lambda i: (i, 0)),
                pl.BlockSpec((eb, h1), lambda i: (i, 0)),
                pl.BlockSpec((eb, h1), lambda i: (i, 0)),
                pl.BlockSpec((eb, 1), lambda i: (i, 0))]
    for a in args[4:]:
        in_specs.append(pl.BlockSpec(a.shape, lambda i: (0, 0)))
    return pl.pallas_call(
        _edge_mlp_body, grid=(e // eb,), in_specs=in_specs,
        out_specs=pl.BlockSpec((eb, emb), lambda i: (i, 0)),
        out_shape=jax.ShapeDtypeStruct((e, emb), F32),
    )(*args)


def _aggr_body(h_ref, s1_ref, s0_ref, c1_ref, c0_ref,
               wh, w1, w0, b1, w2, b2, w3, b3, o_ref):
    hh = h_ref[...]
    sc1 = 1.0 / jnp.maximum(c1_ref[0][:, :1], 1.0)
    sc0 = 1.0 / jnp.maximum(c0_ref[0][:, :1], 1.0)
    u = _dot(hh, wh[...]) + b1[...]
    u += _dot(s1_ref[0] * sc1, w1[...])
    u += _dot(s0_ref[0] * sc0, w0[...])
    u = jnp.maximum(u, 0.0)
    u = jnp.maximum(_dot(u, w2[...]) + b2[...], 0.0)
    o_ref[...] = _dot(u, w3[...]) + b3[...] + hh


def _tc_aggr(h, s_mail, counts, mlp_aggr, bn):
    n, emb = h.shape
    nb = n // bn
    w = mlp_aggr[0]
    args = [h, s_mail, s_mail, counts, counts,
            w[:emb], w[emb:2 * emb], w[2 * emb:],
            mlp_aggr[1].reshape(1, -1),
            mlp_aggr[2], mlp_aggr[3].reshape(1, -1),
            mlp_aggr[4], mlp_aggr[5].reshape(1, -1)]
    in_specs = [pl.BlockSpec((bn, emb), lambda i: (i, 0)),
                pl.BlockSpec((1, bn, emb), lambda i: (0, i, 0)),
                pl.BlockSpec((1, bn, emb), lambda i: (1, i, 0)),
                pl.BlockSpec((1, bn, emb), lambda i: (0, i, 0)),
                pl.BlockSpec((1, bn, emb), lambda i: (1, i, 0))]
    for a in args[5:]:
        in_specs.append(pl.BlockSpec(a.shape, lambda i: (0, 0)))
    return pl.pallas_call(
        _aggr_body, grid=(nb,), in_specs=in_specs,
        out_specs=pl.BlockSpec((bn, emb), lambda i: (i, 0)),
        out_shape=jax.ShapeDtypeStruct((n, emb), F32),
    )(*args)


def _support_body(h_ref, w_ref, o_ref):
    o_ref[...] = _dot(h_ref[...], w_ref[0])


def _tc_support(h, gc_w, bn):
    n, emb = h.shape
    go = gc_w.shape[1]
    d = go // 2
    nb = n // bn
    w_s = jnp.stack([gc_w[:, :d], gc_w[:, d:]])
    return pl.pallas_call(
        _support_body, grid=(2, nb),
        in_specs=[pl.BlockSpec((bn, emb), lambda b, i: (i, 0)),
                  pl.BlockSpec((1, emb, d), lambda b, i: (b, 0, 0))],
        out_specs=pl.BlockSpec((bn, d), lambda b, i: (b * nb + i, 0)),
        out_shape=jax.ShapeDtypeStruct((2 * n, d), F32),
    )(h, w_s)


def _colmax_body(olo_ref, ohi_ref, blo, bhi, mlo_ref, mhi_ref):
    i = pl.program_id(0)

    @pl.when(i == 0)
    def _():
        mlo_ref[...] = jnp.full_like(mlo_ref[...], -jnp.inf)
        mhi_ref[...] = jnp.full_like(mhi_ref[...], -jnp.inf)

    mlo_ref[...] = jnp.maximum(
        mlo_ref[...], jnp.max(olo_ref[0] + blo[...], axis=0, keepdims=True))
    mhi_ref[...] = jnp.maximum(
        mhi_ref[...], jnp.max(ohi_ref[0] + bhi[...], axis=0, keepdims=True))


def _tc_colmax(o_acc, gc_b, bn):
    n = o_acc.shape[1]
    d = o_acc.shape[2]
    return pl.pallas_call(
        _colmax_body, grid=(n // bn,),
        in_specs=[pl.BlockSpec((1, bn, d), lambda i: (0, i, 0)),
                  pl.BlockSpec((1, bn, d), lambda i: (1, i, 0)),
                  pl.BlockSpec((1, d), lambda i: (0, 0)),
                  pl.BlockSpec((1, d), lambda i: (0, 0))],
        out_specs=[pl.BlockSpec((1, d), lambda i: (0, 0)),
                   pl.BlockSpec((1, d), lambda i: (0, 0))],
        out_shape=[jax.ShapeDtypeStruct((1, d), F32)] * 2,
    )(o_acc, o_acc, gc_b[:d].reshape(1, -1), gc_b[d:].reshape(1, -1))


_MESH = dict(core_axis_name="c", subcore_axis_name="s")


def _fill_const(buf, rows, cols, vec16):
    per_row = cols // 16

    def st(i, carry):
        buf[i // per_row, pl.ds((i % per_row) * 16, 16)] = vec16
        return carry

    lax.fori_loop(0, rows * per_row, st, 0)


def _sc_gather2(a_cat, b_cat, idx_a, idx_b):
    e = idx_a.shape[0]
    d = a_cat.shape[1]
    per_w = e // _NW
    ck = 224
    steps = -(-per_w // ck)
    last = per_w - ck

    @functools.partial(
        pl.kernel, mesh=plsc.VectorSubcoreMesh(**_MESH),
        out_type=[jax.ShapeDtypeStruct((e, d), F32)] * 2,
        scratch_types=[pltpu.VMEM((ck,), jnp.int32),
                       pltpu.VMEM((ck,), jnp.int32),
                       pltpu.VMEM((ck, d), F32),
                       pltpu.VMEM((ck, d), F32),
                       pltpu.SemaphoreType.DMA,
                       pltpu.SemaphoreType.DMA],
    )
    def k(a_hbm, b_hbm, ia_hbm, ib_hbm, ga_hbm, gb_hbm,
          ia_v, ib_v, bufa, bufb, sema, semb):
        wid = lax.axis_index("s") * _NC + lax.axis_index("c")
        w0 = wid * per_w

        def body(j, carry):
            base = w0 + jnp.minimum(j * ck, last)
            pltpu.sync_copy(ia_hbm.at[pl.ds(base, ck)], ia_v)
            pltpu.sync_copy(ib_hbm.at[pl.ds(base, ck)], ib_v)
            da = pltpu.async_copy(a_hbm.at[ia_v], bufa, sema)
            db = pltpu.async_copy(b_hbm.at[ib_v], bufb, semb)
            da.wait()
            db.wait()
            pltpu.sync_copy(bufa, ga_hbm.at[pl.ds(base, ck)])
            pltpu.sync_copy(bufb, gb_hbm.at[pl.ds(base, ck)])
            return carry

        lax.fori_loop(0, steps, body, 0)

    return k(a_cat, b_cat, idx_a, idx_b)


def _sc_scatter_rowsplit(data, idx_cat, n):
    e, d = data.shape
    per_t = e // _NS
    ck = 200
    steps = per_t // ck
    zr = ck
    racc = n + _TR
    n_z = racc // zr
    zsteps = -(-n_z // _NS)
    n_ch = n // zr
    wsteps = -(-n_ch // _NS)

    @functools.partial(
        pl.kernel, mesh=plsc.VectorSubcoreMesh(**_MESH),
        out_type=jax.ShapeDtypeStruct((2, n, d), F32),
        scratch_types=[pltpu.VMEM((ck,), jnp.int32),
                       pltpu.VMEM((ck, d), F32),
                       pltpu.VMEM_SHARED((racc, d), F32)],
    )
    def k(m_hbm, idx_hbm, out_hbm, idx_v, dbuf, acc):
        c = lax.axis_index("c")
        s = lax.axis_index("s")
        _fill_const(dbuf, zr, d, jnp.zeros((16,), F32))

        def zc(i, carry):
            cid = s + i * _NS

            @pl.when(cid < n_z)
            def _():
                pltpu.sync_copy(dbuf, acc.at[pl.ds(cid * zr, zr)])

            return carry

        lax.fori_loop(0, zsteps, zc, 0)
        plsc.subcore_barrier()

        def body(j, carry):
            base = s * per_t + j * ck
            pltpu.sync_copy(idx_hbm.at[pl.ds(c * e + base, ck)], idx_v)
            pltpu.sync_copy(m_hbm.at[pl.ds(base, ck)], dbuf)
            pltpu.sync_copy(dbuf, acc.at[idx_v], add=True)
            return carry

        lax.fori_loop(0, steps, body, 0)
        plsc.subcore_barrier()

        def wb(i, carry):
            cid = s + i * _NS

            @pl.when(cid < n_ch)
            def _():
                r0 = cid * zr
                pltpu.sync_copy(acc.at[pl.ds(r0, zr)], dbuf)

                @pl.when(c == 0)
                def _():
                    pltpu.sync_copy(dbuf, out_hbm.at[0, pl.ds(r0, zr)])

                @pl.when(c == 1)
                def _():
                    pltpu.sync_copy(dbuf, out_hbm.at[1, pl.ds(r0, zr)])

            return carry

        lax.fori_loop(0, wsteps, wb, 0)

    return k(data, idx_cat)


def _sc_counts(idx_cat, e, n):
    d = 128
    per_t = e // _NS
    ck = 200
    steps = per_t // ck
    zr = ck
    racc = n + _TR
    n_z = racc // zr
    zsteps = -(-n_z // _NS)
    n_ch = n // zr
    wsteps = -(-n_ch // _NS)

    @functools.partial(
        pl.kernel, mesh=plsc.VectorSubcoreMesh(**_MESH),
        out_type=jax.ShapeDtypeStruct((2, n, d), F32),
        scratch_types=[pltpu.VMEM((ck,), jnp.int32),
                       pltpu.VMEM((ck, d), F32),
                       pltpu.VMEM_SHARED((racc, d), F32)],
    )
    def k(idx_hbm, out_hbm, idx_v, ones_v, acc):
        c = lax.axis_index("c")
        s = lax.axis_index("s")
        _fill_const(ones_v, zr, d, jnp.zeros((16,), F32))

        def zc(i, carry):
            cid = s + i * _NS

            @pl.when(cid < n_z)
            def _():
                pltpu.sync_copy(ones_v, acc.at[pl.ds(cid * zr, zr)])

            return carry

        lax.fori_loop(0, zsteps, zc, 0)
        plsc.subcore_barrier()
        _fill_const(ones_v, ck, d, jnp.ones((16,), F32))

        def body(j, carry):
            base = c * e + s * per_t + j * ck
            pltpu.sync_copy(idx_hbm.at[pl.ds(base, ck)], idx_v)
            pltpu.sync_copy(ones_v, acc.at[idx_v], add=True)
            return carry

        lax.fori_loop(0, steps, body, 0)
        plsc.subcore_barrier()

        def wb(i, carry):
            cid = s + i * _NS

            @pl.when(cid < n_ch)
            def _():
                r0 = cid * zr
                pltpu.sync_copy(acc.at[pl.ds(r0, zr)], ones_v)

                @pl.when(c == 0)
                def _():
                    pltpu.sync_copy(ones_v, out_hbm.at[0, pl.ds(r0, zr)])

                @pl.when(c == 1)
                def _():
                    pltpu.sync_copy(ones_v, out_hbm.at[1, pl.ds(r0, zr)])

            return carry

        lax.fori_loop(0, wsteps, wb, 0)

    return k(idx_cat)


def _sc_spmm(sup_cat, src2, dst, n):
    e = dst.shape[0]
    d = sup_cat.shape[1]
    per_t = e // _NS
    ck = 200
    steps = per_t // ck
    zr = ck
    n_ch = n // zr
    wsteps = -(-n_ch // _NS)

    @functools.partial(
        pl.kernel, mesh=plsc.VectorSubcoreMesh(**_MESH),
        out_type=jax.ShapeDtypeStruct((2, n, d), F32),
        scratch_types=[pltpu.VMEM((ck,), jnp.int32),
                       pltpu.VMEM((ck,), jnp.int32),
                       pltpu.VMEM((ck, d), F32),
                       pltpu.VMEM_SHARED((n, d), F32),
                       pltpu.SemaphoreType.DMA],
    )
    def k(sup_hbm, src_hbm, dst_hbm, out_hbm, is_v, id_v, gbuf, acc, sem):
        c = lax.axis_index("c")
        s = lax.axis_index("s")
        _fill_const(gbuf, zr, d, jnp.zeros((16,), F32))

        def zc(i, carry):
            cid = s + i * _NS

            @pl.when(cid < n_ch)
            def _():
                pltpu.sync_copy(gbuf, acc.at[pl.ds(cid * zr, zr)])

            return carry

        lax.fori_loop(0, wsteps, zc, 0)
        plsc.subcore_barrier()

        def body(j, carry):
            base = s * per_t + j * ck
            pltpu.sync_copy(src_hbm.at[pl.ds(c * e + base, ck)], is_v)
            pltpu.sync_copy(dst_hbm.at[pl.ds(base, ck)], id_v)
            pltpu.async_copy(sup_hbm.at[is_v], gbuf, sem).wait()
            pltpu.sync_copy(gbuf, acc.at[id_v], add=True)
            return carry

        lax.fori_loop(0, steps, body, 0)
        plsc.subcore_barrier()

        def wb(i, carry):
            cid = s + i * _NS

            @pl.when(cid < n_ch)
            def _():
                r0 = cid * zr
                pltpu.sync_copy(acc.at[pl.ds(r0, zr)], gbuf)

                @pl.when(c == 0)
                def _():
                    pltpu.sync_copy(gbuf, out_hbm.at[0, pl.ds(r0, zr)])

                @pl.when(c == 1)
                def _():
                    pltpu.sync_copy(gbuf, out_hbm.at[1, pl.ds(r0, zr)])

            return carry

        lax.fori_loop(0, wsteps, wb, 0)

    return k(sup_cat, src2, dst)


def kernel(x, l_e, edge_index, edge_label, mlp_v, mlp_e, mlp_edge1,
           mlp_edge0, mlp_aggr, gc_w, gc_b):
    n, emb = x.shape
    e = l_e.shape[0]
    bn = 2000
    eb = 1000
    src = edge_index[0].astype(jnp.int32)
    dst = edge_index[1].astype(jnp.int32)
    mask_f = (edge_label == 1).astype(F32)[:, None]
    off = jnp.where(edge_label == 1, 0, n).astype(jnp.int32)
    src_adj = src + off
    dst_adj = dst + off
    trash = (n + dst % _TR).astype(jnp.int32)
    idx_mail = jnp.concatenate([
        jnp.where(edge_label == 1, dst, trash),
        jnp.where(edge_label == 0, dst, trash)]).astype(jnp.int32)
    src2 = jnp.concatenate([src, src + n]).astype(jnp.int32)

    h = _tc_mlp3(x, mlp_v, bn)
    c_sel = _tc_edge_embed(l_e, mask_f, mlp_e, mlp_edge1, mlp_edge0, eb)
    counts = _sc_counts(idx_mail, e, n)

    wa_s = jnp.stack([mlp_edge1[0][:emb], mlp_edge0[0][:emb]])
    wb_s = jnp.stack([mlp_edge1[0][emb:2 * emb], mlp_edge0[0][emb:2 * emb]])

    for _ in range(2):
        a_cat, b_cat = _tc_tables(h, wa_s, wb_s, bn)
        ga, gb = _sc_gather2(a_cat, b_cat, src_adj, dst_adj)
        m = _tc_edge_mlp(ga, gb, c_sel, mask_f, mlp_edge1, mlp_edge0, eb)
        s_mail = _sc_scatter_rowsplit(m, idx_mail, n)
        h = _tc_aggr(h, s_mail, counts, mlp_aggr, bn)

    sup_cat = _tc_support(h, gc_w, bn)
    o_acc = _sc_spmm(sup_cat, src2, dst, n)
    mlo, mhi = _tc_colmax(o_acc, gc_b, bn)
    return jnp.concatenate([mlo[0], mhi[0]], axis=0)


# speedup vs baseline: 2.9073x; 1.1309x over previous
import functools

import jax
import jax.numpy as jnp
from jax import lax
from jax.experimental import pallas as pl
from jax.experimental.pallas import tpu as pltpu
from jax.experimental.pallas import tpu_sc as plsc

F32 = jnp.float32
_NC, _NS = 2, 16
_NW = _NC * _NS
_TR = 800


def _dot(a, b):
    return jnp.dot(a.astype(jnp.bfloat16), b.astype(jnp.bfloat16),
                   preferred_element_type=F32)


def _pack_bf16(even_f32, odd_f32):
    ae = lax.bitcast_convert_type(
        even_f32.astype(jnp.bfloat16).astype(F32), jnp.int32)
    ao = lax.bitcast_convert_type(
        odd_f32.astype(jnp.bfloat16).astype(F32), jnp.int32)
    return lax.shift_right_logical(ae, 16) | ao


def _unpack_bf16(xi):
    lo = lax.bitcast_convert_type(lax.shift_left(xi, 16), F32)
    hi = lax.bitcast_convert_type(xi & jnp.int32(-65536), F32)
    return lo, hi


def _mlp3_body(x_ref, w1, b1, w2, b2, w3, b3, o_ref):
    h = jnp.maximum(_dot(x_ref[...], w1[...]) + b1[...], 0.0)
    h = jnp.maximum(_dot(h, w2[...]) + b2[...], 0.0)
    o_ref[...] = _dot(h, w3[...]) + b3[...]


def _tc_mlp3(x, p, bn):
    n, din = x.shape
    dout = p[4].shape[1]
    args = [x, p[0], p[1].reshape(1, -1), p[2], p[3].reshape(1, -1),
            p[4], p[5].reshape(1, -1)]
    in_specs = [pl.BlockSpec((bn, din), lambda i: (i, 0))]
    for a in args[1:]:
        in_specs.append(pl.BlockSpec(a.shape, lambda i: (0, 0)))
    return pl.pallas_call(
        _mlp3_body, grid=(n // bn,), in_specs=in_specs,
        out_specs=pl.BlockSpec((bn, dout), lambda i: (i, 0)),
        out_shape=jax.ShapeDtypeStruct((n, dout), F32),
    )(*args)


def _edge_embed_body(le_ref, mk_ref, w1, b1, w2, b2, w3, b3,
                     wc1e, wc1o, bc1e, bc1o, wc0e, wc0o, bc0e, bc0o, c_ref):
    h = jnp.maximum(le_ref[...] * w1[...] + b1[...], 0.0)
    h = jnp.maximum(_dot(h, w2[...]) + b2[...], 0.0)
    h = _dot(h, w3[...]) + b3[...]
    mk = mk_ref[...]
    c1e = _dot(h, wc1e[...]) + bc1e[...]
    c0e = _dot(h, wc0e[...]) + bc0e[...]
    c1o = _dot(h, wc1o[...]) + bc1o[...]
    c0o = _dot(h, wc0o[...]) + bc0o[...]
    ce = c0e + mk * (c1e - c0e)
    co = c0o + mk * (c1o - c0o)
    c_ref[...] = _pack_bf16(ce, co)


def _tc_edge_embed(l_e, mask_f, mlp_e, mlp_edge1, mlp_edge0, eb):
    e = l_e.shape[0]
    emb = mlp_e[4].shape[1]
    wc1 = mlp_edge1[0][2 * emb:]
    wc0 = mlp_edge0[0][2 * emb:]
    args = [l_e, mask_f,
            mlp_e[0], mlp_e[1].reshape(1, -1),
            mlp_e[2], mlp_e[3].reshape(1, -1),
            mlp_e[4], mlp_e[5].reshape(1, -1),
            wc1[:, 0::2], wc1[:, 1::2],
            mlp_edge1[1][0::2].reshape(1, -1), mlp_edge1[1][1::2].reshape(1, -1),
            wc0[:, 0::2], wc0[:, 1::2],
            mlp_edge0[1][0::2].reshape(1, -1), mlp_edge0[1][1::2].reshape(1, -1)]
    h1 = mlp_edge1[0].shape[1]
    in_specs = [pl.BlockSpec((eb, 1), lambda i: (i, 0)),
                pl.BlockSpec((eb, 1), lambda i: (i, 0))]
    for a in args[2:]:
        in_specs.append(pl.BlockSpec(a.shape, lambda i: (0, 0)))
    return pl.pallas_call(
        _edge_embed_body, grid=(e // eb,), in_specs=in_specs,
        out_specs=pl.BlockSpec((eb, h1 // 2), lambda i: (i, 0)),
        out_shape=jax.ShapeDtypeStruct((e, h1 // 2), jnp.int32),
    )(*args)


def _tables_body(h_ref, wae, wao, wbe, wbo, a_ref, b_ref):
    hh = h_ref[...]
    a_ref[...] = _pack_bf16(_dot(hh, wae[0]), _dot(hh, wao[0]))
    b_ref[...] = _pack_bf16(_dot(hh, wbe[0]), _dot(hh, wbo[0]))


def _tc_tables(h, w4, bn):
    n, k = h.shape
    m = w4[0].shape[2]
    nb = n // bn
    in_specs = [pl.BlockSpec((bn, k), lambda b, i: (i, 0))]
    for _ in range(4):
        in_specs.append(pl.BlockSpec((1, k, m), lambda b, i: (b, 0, 0)))
    return pl.pallas_call(
        _tables_body, grid=(2, nb), in_specs=in_specs,
        out_specs=[pl.BlockSpec((bn, m), lambda b, i: (b * nb + i, 0)),
                   pl.BlockSpec((bn, m), lambda b, i: (b * nb + i, 0))],
        out_shape=[jax.ShapeDtypeStruct((2 * n, m), jnp.int32)] * 2,
    )(h, *w4)


def _edge_mlp_body(ga_ref, gb_ref, c_ref, mk_ref,
                   w21e, w21o, b21, w31, b31,
                   w20e, w20o, b20, w30, b30, m_ref):
    gae, gao = _unpack_bf16(ga_ref[...])
    gbe, gbo = _unpack_bf16(gb_ref[...])
    ce, co = _unpack_bf16(c_ref[...])
    ze = jnp.maximum(gae + gbe + ce, 0.0)
    zo = jnp.maximum(gao + gbo + co, 0.0)
    u1 = jnp.maximum(_dot(ze, w21e[...]) + _dot(zo, w21o[...]) + b21[...], 0.0)
    m1 = _dot(u1, w31[...]) + b31[...]
    u0 = jnp.maximum(_dot(ze, w20e[...]) + _dot(zo, w20o[...]) + b20[...], 0.0)
    m0 = _dot(u0, w30[...]) + b30[...]
    m_ref[...] = m0 + mk_ref[...] * (m1 - m0)


def _tc_edge_mlp(ga, gb, c_sel, mask_f, mlp_edge1, mlp_edge0, eb):
    e, hp = ga.shape
    emb = mlp_edge1[4].shape[1]
    args = [ga, gb, c_sel, mask_f,
            mlp_edge1[2][0::2], mlp_edge1[2][1::2],
            mlp_edge1[3].reshape(1, -1),
            mlp_edge1[4], mlp_edge1[5].reshape(1, -1),
            mlp_edge0[2][0::2], mlp_edge0[2][1::2],
            mlp_edge0[3].reshape(1, -1),
            mlp_edge0[4], mlp_edge0[5].reshape(1, -1)]
    in_specs = [pl.BlockSpec((eb, hp), lambda i: (i, 0)),
                pl.BlockSpec((eb, hp), lambda i: (i, 0)),
                pl.BlockSpec((eb, hp), lambda i: (i, 0)),
                pl.BlockSpec((eb, 1), lambda i: (i, 0))]
    for a in args[4:]:
        in_specs.append(pl.BlockSpec(a.shape, lambda i: (0, 0)))
    return pl.pallas_call(
        _edge_mlp_body, grid=(e // eb,), in_specs=in_specs,
        out_specs=pl.BlockSpec((eb, emb), lambda i: (i, 0)),
        out_shape=jax.ShapeDtypeStruct((e, emb), F32),
    )(*args)


def _aggr_body(h_ref, s1_ref, s0_ref, c1_ref, c0_ref,
               wh, w1, w0, b1, w2, b2, w3, b3, o_ref):
    hh = h_ref[...]
    sc1 = 1.0 / jnp.maximum(c1_ref[0][:, :1], 1.0)
    sc0 = 1.0 / jnp.maximum(c0_ref[0][:, :1], 1.0)
    u = _dot(hh, wh[...]) + b1[...]
    u += _dot(s1_ref[0] * sc1, w1[...])
    u += _dot(s0_ref[0] * sc0, w0[...])
    u = jnp.maximum(u, 0.0)
    u = jnp.maximum(_dot(u, w2[...]) + b2[...], 0.0)
    o_ref[...] = _dot(u, w3[...]) + b3[...] + hh


def _tc_aggr(h, s_mail, counts, mlp_aggr, bn):
    n, emb = h.shape
    nb = n // bn
    w = mlp_aggr[0]
    args = [h, s_mail, s_mail, counts, counts,
            w[:emb], w[emb:2 * emb], w[2 * emb:],
            mlp_aggr[1].reshape(1, -1),
            mlp_aggr[2], mlp_aggr[3].reshape(1, -1),
            mlp_aggr[4], mlp_aggr[5].reshape(1, -1)]
    in_specs = [pl.BlockSpec((bn, emb), lambda i: (i, 0)),
                pl.BlockSpec((1, bn, emb), lambda i: (0, i, 0)),
                pl.BlockSpec((1, bn, emb), lambda i: (1, i, 0)),
                pl.BlockSpec((1, bn, emb), lambda i: (0, i, 0)),
                pl.BlockSpec((1, bn, emb), lambda i: (1, i, 0))]
    for a in args[5:]:
        in_specs.append(pl.BlockSpec(a.shape, lambda i: (0, 0)))
    return pl.pallas_call(
        _aggr_body, grid=(nb,), in_specs=in_specs,
        out_specs=pl.BlockSpec((bn, emb), lambda i: (i, 0)),
        out_shape=jax.ShapeDtypeStruct((n, emb), F32),
    )(*args)


def _support_body(h_ref, w_ref, o_ref):
    o_ref[...] = _dot(h_ref[...], w_ref[0])


def _tc_support(h, gc_w, bn):
    n, emb = h.shape
    go = gc_w.shape[1]
    d = go // 2
    nb = n // bn
    w_s = jnp.stack([gc_w[:, :d], gc_w[:, d:]])
    return pl.pallas_call(
        _support_body, grid=(2, nb),
        in_specs=[pl.BlockSpec((bn, emb), lambda b, i: (i, 0)),
                  pl.BlockSpec((1, emb, d), lambda b, i: (b, 0, 0))],
        out_specs=pl.BlockSpec((bn, d), lambda b, i: (b * nb + i, 0)),
        out_shape=jax.ShapeDtypeStruct((2 * n, d), F32),
    )(h, w_s)


def _colmax_body(olo_ref, ohi_ref, blo, bhi, mlo_ref, mhi_ref):
    i = pl.program_id(0)

    @pl.when(i == 0)
    def _():
        mlo_ref[...] = jnp.full_like(mlo_ref[...], -jnp.inf)
        mhi_ref[...] = jnp.full_like(mhi_ref[...], -jnp.inf)

    mlo_ref[...] = jnp.maximum(
        mlo_ref[...], jnp.max(olo_ref[0] + blo[...], axis=0, keepdims=True))
    mhi_ref[...] = jnp.maximum(
        mhi_ref[...], jnp.max(ohi_ref[0] + bhi[...], axis=0, keepdims=True))


def _tc_colmax(o_acc, gc_b, bn):
    n = o_acc.shape[1]
    d = o_acc.shape[2]
    return pl.pallas_call(
        _colmax_body, grid=(n // bn,),
        in_specs=[pl.BlockSpec((1, bn, d), lambda i: (0, i, 0)),
                  pl.BlockSpec((1, bn, d), lambda i: (1, i, 0)),
                  pl.BlockSpec((1, d), lambda i: (0, 0)),
                  pl.BlockSpec((1, d), lambda i: (0, 0))],
        out_specs=[pl.BlockSpec((1, d), lambda i: (0, 0)),
                   pl.BlockSpec((1, d), lambda i: (0, 0))],
        out_shape=[jax.ShapeDtypeStruct((1, d), F32)] * 2,
    )(o_acc, o_acc, gc_b[:d].reshape(1, -1), gc_b[d:].reshape(1, -1))


_MESH = dict(core_axis_name="c", subcore_axis_name="s")


def _fill_const(buf, rows, cols, vec16):
    per_row = cols // 16

    def st(i, carry):
        buf[i // per_row, pl.ds((i % per_row) * 16, 16)] = vec16
        return carry

    lax.fori_loop(0, rows * per_row, st, 0)


def _sc_gather2(a_cat, b_cat, idx_a, idx_b):
    e = idx_a.shape[0]
    d = a_cat.shape[1]
    per_w = e // _NW
    ck = 400
    steps = -(-per_w // ck)
    last = per_w - ck

    @functools.partial(
        pl.kernel, mesh=plsc.VectorSubcoreMesh(**_MESH),
        out_type=[jax.ShapeDtypeStruct((e, d), jnp.int32)] * 2,
        scratch_types=[pltpu.VMEM((ck,), jnp.int32),
                       pltpu.VMEM((ck,), jnp.int32),
                       pltpu.VMEM((ck, d), jnp.int32),
                       pltpu.VMEM((ck, d), jnp.int32),
                       pltpu.SemaphoreType.DMA,
                       pltpu.SemaphoreType.DMA],
    )
    def k(a_hbm, b_hbm, ia_hbm, ib_hbm, ga_hbm, gb_hbm,
          ia_v, ib_v, bufa, bufb, sema, semb):
        wid = lax.axis_index("s") * _NC + lax.axis_index("c")
        w0 = wid * per_w

        def body(j, carry):
            base = w0 + jnp.minimum(j * ck, last)
            pltpu.sync_copy(ia_hbm.at[pl.ds(base, ck)], ia_v)
            pltpu.sync_copy(ib_hbm.at[pl.ds(base, ck)], ib_v)
            da = pltpu.async_copy(a_hbm.at[ia_v], bufa, sema)
            db = pltpu.async_copy(b_hbm.at[ib_v], bufb, semb)
            da.wait()
            db.wait()
            pltpu.sync_copy(bufa, ga_hbm.at[pl.ds(base, ck)])
            pltpu.sync_copy(bufb, gb_hbm.at[pl.ds(base, ck)])
            return carry

        lax.fori_loop(0, steps, body, 0)

    return k(a_cat, b_cat, idx_a, idx_b)


def _sc_scatter_rowsplit(data, idx_cat, n):
    e, d = data.shape
    per_t = e // _NS
    ck = 200
    steps = per_t // ck
    zr = ck
    racc = n + _TR
    n_z = racc // zr
    zsteps = -(-n_z // _NS)
    n_ch = n // zr
    wsteps = -(-n_ch // _NS)

    @functools.partial(
        pl.kernel, mesh=plsc.VectorSubcoreMesh(**_MESH),
        out_type=jax.ShapeDtypeStruct((2, n, d), F32),
        scratch_types=[pltpu.VMEM((ck,), jnp.int32),
                       pltpu.VMEM((ck, d), F32),
                       pltpu.VMEM_SHARED((racc, d), F32)],
    )
    def k(m_hbm, idx_hbm, out_hbm, idx_v, dbuf, acc):
        c = lax.axis_index("c")
        s = lax.axis_index("s")
        _fill_const(dbuf, zr, d, jnp.zeros((16,), F32))

        def zc(i, carry):
            cid = s + i * _NS

            @pl.when(cid < n_z)
            def _():
                pltpu.sync_copy(dbuf, acc.at[pl.ds(cid * zr, zr)])

            return carry

        lax.fori_loop(0, zsteps, zc, 0)
        plsc.subcore_barrier()

        def body(j, carry):
            base = s * per_t + j * ck
            pltpu.sync_copy(idx_hbm.at[pl.ds(c * e + base, ck)], idx_v)
            pltpu.sync_copy(m_hbm.at[pl.ds(base, ck)], dbuf)
            pltpu.sync_copy(dbuf, acc.at[idx_v], add=True)
            return carry

        lax.fori_loop(0, steps, body, 0)
        plsc.subcore_barrier()

        def wb(i, carry):
            cid = s + i * _NS

            @pl.when(cid < n_ch)
            def _():
                r0 = cid * zr
                pltpu.sync_copy(acc.at[pl.ds(r0, zr)], dbuf)

                @pl.when(c == 0)
                def _():
                    pltpu.sync_copy(dbuf, out_hbm.at[0, pl.ds(r0, zr)])

                @pl.when(c == 1)
                def _():
                    pltpu.sync_copy(dbuf, out_hbm.at[1, pl.ds(r0, zr)])

            return carry

        lax.fori_loop(0, wsteps, wb, 0)

    return k(data, idx_cat)


def _sc_counts(idx_cat, e, n):
    d = 128
    per_t = e // _NS
    ck = 200
    steps = per_t // ck
    zr = ck
    racc = n + _TR
    n_z = racc // zr
    zsteps = -(-n_z // _NS)
    n_ch = n // zr
    wsteps = -(-n_ch // _NS)

    @functools.partial(
        pl.kernel, mesh=plsc.VectorSubcoreMesh(**_MESH),
        out_type=jax.ShapeDtypeStruct((2, n, d), F32),
        scratch_types=[pltpu.VMEM((ck,), jnp.int32),
                       pltpu.VMEM((ck, d), F32),
                       pltpu.VMEM_SHARED((racc, d), F32)],
    )
    def k(idx_hbm, out_hbm, idx_v, ones_v, acc):
        c = lax.axis_index("c")
        s = lax.axis_index("s")
        _fill_const(ones_v, zr, d, jnp.zeros((16,), F32))

        def zc(i, carry):
            cid = s + i * _NS

            @pl.when(cid < n_z)
            def _():
                pltpu.sync_copy(ones_v, acc.at[pl.ds(cid * zr, zr)])

            return carry

        lax.fori_loop(0, zsteps, zc, 0)
        plsc.subcore_barrier()
        _fill_const(ones_v, ck, d, jnp.ones((16,), F32))

        def body(j, carry):
            base = c * e + s * per_t + j * ck
            pltpu.sync_copy(idx_hbm.at[pl.ds(base, ck)], idx_v)
            pltpu.sync_copy(ones_v, acc.at[idx_v], add=True)
            return carry

        lax.fori_loop(0, steps, body, 0)
        plsc.subcore_barrier()

        def wb(i, carry):
            cid = s + i * _NS

            @pl.when(cid < n_ch)
            def _():
                r0 = cid * zr
                pltpu.sync_copy(acc.at[pl.ds(r0, zr)], ones_v)

                @pl.when(c == 0)
                def _():
                    pltpu.sync_copy(ones_v, out_hbm.at[0, pl.ds(r0, zr)])

                @pl.when(c == 1)
                def _():
                    pltpu.sync_copy(ones_v, out_hbm.at[1, pl.ds(r0, zr)])

            return carry

        lax.fori_loop(0, wsteps, wb, 0)

    return k(idx_cat)


def _sc_spmm(sup_cat, src2, dst, n):
    e = dst.shape[0]
    d = sup_cat.shape[1]
    per_t = e // _NS
    ck = 200
    steps = per_t // ck
    zr = ck
    n_ch = n // zr
    wsteps = -(-n_ch // _NS)

    @functools.partial(
        pl.kernel, mesh=plsc.VectorSubcoreMesh(**_MESH),
        out_type=jax.ShapeDtypeStruct((2, n, d), F32),
        scratch_types=[pltpu.VMEM((ck,), jnp.int32),
                       pltpu.VMEM((ck,), jnp.int32),
                       pltpu.VMEM((ck, d), F32),
                       pltpu.VMEM_SHARED((n, d), F32),
                       pltpu.SemaphoreType.DMA],
    )
    def k(sup_hbm, src_hbm, dst_hbm, out_hbm, is_v, id_v, gbuf, acc, sem):
        c = lax.axis_index("c")
        s = lax.axis_index("s")
        _fill_const(gbuf, zr, d, jnp.zeros((16,), F32))

        def zc(i, carry):
            cid = s + i * _NS

            @pl.when(cid < n_ch)
            def _():
                pltpu.sync_copy(gbuf, acc.at[pl.ds(cid * zr, zr)])

            return carry

        lax.fori_loop(0, wsteps, zc, 0)
        plsc.subcore_barrier()

        def body(j, carry):
            base = s * per_t + j * ck
            pltpu.sync_copy(src_hbm.at[pl.ds(c * e + base, ck)], is_v)
            pltpu.sync_copy(dst_hbm.at[pl.ds(base, ck)], id_v)
            pltpu.async_copy(sup_hbm.at[is_v], gbuf, sem).wait()
            pltpu.sync_copy(gbuf, acc.at[id_v], add=True)
            return carry

        lax.fori_loop(0, steps, body, 0)
        plsc.subcore_barrier()

        def wb(i, carry):
            cid = s + i * _NS

            @pl.when(cid < n_ch)
            def _():
                r0 = cid * zr
                pltpu.sync_copy(acc.at[pl.ds(r0, zr)], gbuf)

                @pl.when(c == 0)
                def _():
                    pltpu.sync_copy(gbuf, out_hbm.at[0, pl.ds(r0, zr)])

                @pl.when(c == 1)
                def _():
                    pltpu.sync_copy(gbuf, out_hbm.at[1, pl.ds(r0, zr)])

            return carry

        lax.fori_loop(0, wsteps, wb, 0)

    return k(sup_cat, src2, dst)


def kernel(x, l_e, edge_index, edge_label, mlp_v, mlp_e, mlp_edge1,
           mlp_edge0, mlp_aggr, gc_w, gc_b):
    n, emb = x.shape
    e = l_e.shape[0]
    bn = 2000
    eb = 1000
    src = edge_index[0].astype(jnp.int32)
    dst = edge_index[1].astype(jnp.int32)
    mask_f = (edge_label == 1).astype(F32)[:, None]
    off = jnp.where(edge_label == 1, 0, n).astype(jnp.int32)
    src_adj = src + off
    dst_adj = dst + off
    trash = (n + dst % _TR).astype(jnp.int32)
    idx_mail = jnp.concatenate([
        jnp.where(edge_label == 1, dst, trash),
        jnp.where(edge_label == 0, dst, trash)]).astype(jnp.int32)
    src2 = jnp.concatenate([src, src + n]).astype(jnp.int32)

    h = _tc_mlp3(x, mlp_v, bn)
    c_sel = _tc_edge_embed(l_e, mask_f, mlp_e, mlp_edge1, mlp_edge0, eb)
    counts = _sc_counts(idx_mail, e, n)

    wa1 = mlp_edge1[0][:emb]
    wa0 = mlp_edge0[0][:emb]
    wb1 = mlp_edge1[0][emb:2 * emb]
    wb0 = mlp_edge0[0][emb:2 * emb]
    w4 = [jnp.stack([wa1[:, 0::2], wa0[:, 0::2]]),
          jnp.stack([wa1[:, 1::2], wa0[:, 1::2]]),
          jnp.stack([wb1[:, 0::2], wb0[:, 0::2]]),
          jnp.stack([wb1[:, 1::2], wb0[:, 1::2]])]

    for _ in range(2):
        a_cat, b_cat = _tc_tables(h, w4, bn)
        ga, gb = _sc_gather2(a_cat, b_cat, src_adj, dst_adj)
        m = _tc_edge_mlp(ga, gb, c_sel, mask_f, mlp_edge1, mlp_edge0, eb)
        s_mail = _sc_scatter_rowsplit(m, idx_mail, n)
        h = _tc_aggr(h, s_mail, counts, mlp_aggr, bn)

    sup_cat = _tc_support(h, gc_w, bn)
    o_acc = _sc_spmm(sup_cat, src2, dst, n)
    mlo, mhi = _tc_colmax(o_acc, gc_b, bn)
    return jnp.concatenate([mlo[0], mhi[0]], axis=0)


# speedup vs baseline: 2.9259x; 1.0064x over previous
import functools

import jax
import jax.numpy as jnp
from jax import lax
from jax.experimental import pallas as pl
from jax.experimental.pallas import tpu as pltpu
from jax.experimental.pallas import tpu_sc as plsc

F32 = jnp.float32
_NC, _NS = 2, 16
_NW = _NC * _NS
_TR = 800


def _dot(a, b):
    return jnp.dot(a.astype(jnp.bfloat16), b.astype(jnp.bfloat16),
                   preferred_element_type=F32)


def _pack_bf16(even_f32, odd_f32):
    ae = lax.bitcast_convert_type(
        even_f32.astype(jnp.bfloat16).astype(F32), jnp.int32)
    ao = lax.bitcast_convert_type(
        odd_f32.astype(jnp.bfloat16).astype(F32), jnp.int32)
    return lax.shift_right_logical(ae, 16) | ao


def _unpack_bf16(xi):
    lo = lax.bitcast_convert_type(lax.shift_left(xi, 16), F32)
    hi = lax.bitcast_convert_type(xi & jnp.int32(-65536), F32)
    return lo, hi


def _mlp3_body(x_ref, w1, b1, w2, b2, w3, b3, o_ref):
    h = jnp.maximum(_dot(x_ref[...], w1[...]) + b1[...], 0.0)
    h = jnp.maximum(_dot(h, w2[...]) + b2[...], 0.0)
    o_ref[...] = _dot(h, w3[...]) + b3[...]


def _tc_mlp3(x, p, bn):
    n, din = x.shape
    dout = p[4].shape[1]
    args = [x, p[0], p[1].reshape(1, -1), p[2], p[3].reshape(1, -1),
            p[4], p[5].reshape(1, -1)]
    in_specs = [pl.BlockSpec((bn, din), lambda i: (i, 0))]
    for a in args[1:]:
        in_specs.append(pl.BlockSpec(a.shape, lambda i: (0, 0)))
    return pl.pallas_call(
        _mlp3_body, grid=(n // bn,), in_specs=in_specs,
        out_specs=pl.BlockSpec((bn, dout), lambda i: (i, 0)),
        out_shape=jax.ShapeDtypeStruct((n, dout), F32),
    )(*args)


def _edge_embed_body(le_ref, mk_ref, w1, b1, w2, b2, w3, b3,
                     wc1e, wc1o, bc1e, bc1o, wc0e, wc0o, bc0e, bc0o, c_ref):
    h = jnp.maximum(le_ref[...] * w1[...] + b1[...], 0.0)
    h = jnp.maximum(_dot(h, w2[...]) + b2[...], 0.0)
    h = _dot(h, w3[...]) + b3[...]
    mk = mk_ref[...]
    c1e = _dot(h, wc1e[...]) + bc1e[...]
    c0e = _dot(h, wc0e[...]) + bc0e[...]
    c1o = _dot(h, wc1o[...]) + bc1o[...]
    c0o = _dot(h, wc0o[...]) + bc0o[...]
    ce = c0e + mk * (c1e - c0e)
    co = c0o + mk * (c1o - c0o)
    c_ref[...] = _pack_bf16(ce, co)


def _tc_edge_embed(l_e, mask_f, mlp_e, mlp_edge1, mlp_edge0, eb):
    e = l_e.shape[0]
    emb = mlp_e[4].shape[1]
    wc1 = mlp_edge1[0][2 * emb:]
    wc0 = mlp_edge0[0][2 * emb:]
    args = [l_e, mask_f,
            mlp_e[0], mlp_e[1].reshape(1, -1),
            mlp_e[2], mlp_e[3].reshape(1, -1),
            mlp_e[4], mlp_e[5].reshape(1, -1),
            wc1[:, 0::2], wc1[:, 1::2],
            mlp_edge1[1][0::2].reshape(1, -1), mlp_edge1[1][1::2].reshape(1, -1),
            wc0[:, 0::2], wc0[:, 1::2],
            mlp_edge0[1][0::2].reshape(1, -1), mlp_edge0[1][1::2].reshape(1, -1)]
    h1 = mlp_edge1[0].shape[1]
    in_specs = [pl.BlockSpec((eb, 1), lambda i: (i, 0)),
                pl.BlockSpec((eb, 1), lambda i: (i, 0))]
    for a in args[2:]:
        in_specs.append(pl.BlockSpec(a.shape, lambda i: (0, 0)))
    return pl.pallas_call(
        _edge_embed_body, grid=(e // eb,), in_specs=in_specs,
        out_specs=pl.BlockSpec((eb, h1 // 2), lambda i: (i, 0)),
        out_shape=jax.ShapeDtypeStruct((e, h1 // 2), jnp.int32),
    )(*args)


def _tables_body(h_ref, wae, wao, wbe, wbo, a_ref, b_ref):
    hh = h_ref[...]
    a_ref[...] = _pack_bf16(_dot(hh, wae[0]), _dot(hh, wao[0]))
    b_ref[...] = _pack_bf16(_dot(hh, wbe[0]), _dot(hh, wbo[0]))


def _tc_tables(h, w4, bn):
    n, k = h.shape
    m = w4[0].shape[2]
    nb = n // bn
    in_specs = [pl.BlockSpec((bn, k), lambda b, i: (i, 0))]
    for _ in range(4):
        in_specs.append(pl.BlockSpec((1, k, m), lambda b, i: (b, 0, 0)))
    return pl.pallas_call(
        _tables_body, grid=(2, nb), in_specs=in_specs,
        out_specs=[pl.BlockSpec((bn, m), lambda b, i: (b * nb + i, 0)),
                   pl.BlockSpec((bn, m), lambda b, i: (b * nb + i, 0))],
        out_shape=[jax.ShapeDtypeStruct((2 * n, m), jnp.int32)] * 2,
    )(h, *w4)


def _edge_mlp_body(ga_ref, gb_ref, c_ref, mk_ref,
                   w21e, w21o, b21, w31, b31,
                   w20e, w20o, b20, w30, b30, m_ref):
    gae, gao = _unpack_bf16(ga_ref[...])
    gbe, gbo = _unpack_bf16(gb_ref[...])
    ce, co = _unpack_bf16(c_ref[...])
    ze = jnp.maximum(gae + gbe + ce, 0.0)
    zo = jnp.maximum(gao + gbo + co, 0.0)
    u1 = jnp.maximum(_dot(ze, w21e[...]) + _dot(zo, w21o[...]) + b21[...], 0.0)
    m1 = _dot(u1, w31[...]) + b31[...]
    u0 = jnp.maximum(_dot(ze, w20e[...]) + _dot(zo, w20o[...]) + b20[...], 0.0)
    m0 = _dot(u0, w30[...]) + b30[...]
    m_ref[...] = m0 + mk_ref[...] * (m1 - m0)


def _tc_edge_mlp(ga, gb, c_sel, mask_f, mlp_edge1, mlp_edge0, eb):
    e, hp = ga.shape
    emb = mlp_edge1[4].shape[1]
    args = [ga, gb, c_sel, mask_f,
            mlp_edge1[2][0::2], mlp_edge1[2][1::2],
            mlp_edge1[3].reshape(1, -1),
            mlp_edge1[4], mlp_edge1[5].reshape(1, -1),
            mlp_edge0[2][0::2], mlp_edge0[2][1::2],
            mlp_edge0[3].reshape(1, -1),
            mlp_edge0[4], mlp_edge0[5].reshape(1, -1)]
    in_specs = [pl.BlockSpec((eb, hp), lambda i: (i, 0)),
                pl.BlockSpec((eb, hp), lambda i: (i, 0)),
                pl.BlockSpec((eb, hp), lambda i: (i, 0)),
                pl.BlockSpec((eb, 1), lambda i: (i, 0))]
    for a in args[4:]:
        in_specs.append(pl.BlockSpec(a.shape, lambda i: (0, 0)))
    return pl.pallas_call(
        _edge_mlp_body, grid=(e // eb,), in_specs=in_specs,
        out_specs=pl.BlockSpec((eb, emb), lambda i: (i, 0)),
        out_shape=jax.ShapeDtypeStruct((e, emb), F32),
    )(*args)


def _aggr_body(h_ref, s1_ref, s0_ref, c1_ref, c0_ref,
               wh, w1, w0, b1, w2, b2, w3, b3, o_ref):
    hh = h_ref[...]
    sc1 = 1.0 / jnp.maximum(c1_ref[0][:, :1], 1.0)
    sc0 = 1.0 / jnp.maximum(c0_ref[0][:, :1], 1.0)
    u = _dot(hh, wh[...]) + b1[...]
    u += _dot(s1_ref[0] * sc1, w1[...])
    u += _dot(s0_ref[0] * sc0, w0[...])
    u = jnp.maximum(u, 0.0)
    u = jnp.maximum(_dot(u, w2[...]) + b2[...], 0.0)
    o_ref[...] = _dot(u, w3[...]) + b3[...] + hh


def _tc_aggr(h, s_mail, counts, mlp_aggr, bn):
    n, emb = h.shape
    nb = n // bn
    w = mlp_aggr[0]
    args = [h, s_mail, s_mail, counts, counts,
            w[:emb], w[emb:2 * emb], w[2 * emb:],
            mlp_aggr[1].reshape(1, -1),
            mlp_aggr[2], mlp_aggr[3].reshape(1, -1),
            mlp_aggr[4], mlp_aggr[5].reshape(1, -1)]
    in_specs = [pl.BlockSpec((bn, emb), lambda i: (i, 0)),
                pl.BlockSpec((1, bn, emb), lambda i: (0, i, 0)),
                pl.BlockSpec((1, bn, emb), lambda i: (1, i, 0)),
                pl.BlockSpec((1, bn, emb), lambda i: (0, i, 0)),
                pl.BlockSpec((1, bn, emb), lambda i: (1, i, 0))]
    for a in args[5:]:
        in_specs.append(pl.BlockSpec(a.shape, lambda i: (0, 0)))
    return pl.pallas_call(
        _aggr_body, grid=(nb,), in_specs=in_specs,
        out_specs=pl.BlockSpec((bn, emb), lambda i: (i, 0)),
        out_shape=jax.ShapeDtypeStruct((n, emb), F32),
    )(*args)


def _support_body(h_ref, w_ref, o_ref):
    o_ref[...] = _dot(h_ref[...], w_ref[0])


def _tc_support(h, gc_w, bn):
    n, emb = h.shape
    go = gc_w.shape[1]
    d = go // 2
    nb = n // bn
    w_s = jnp.stack([gc_w[:, :d], gc_w[:, d:]])
    return pl.pallas_call(
        _support_body, grid=(2, nb),
        in_specs=[pl.BlockSpec((bn, emb), lambda b, i: (i, 0)),
                  pl.BlockSpec((1, emb, d), lambda b, i: (b, 0, 0))],
        out_specs=pl.BlockSpec((bn, d), lambda b, i: (b * nb + i, 0)),
        out_shape=jax.ShapeDtypeStruct((2 * n, d), F32),
    )(h, w_s)


def _colmax_body(olo_ref, ohi_ref, blo, bhi, mlo_ref, mhi_ref):
    i = pl.program_id(0)

    @pl.when(i == 0)
    def _():
        mlo_ref[...] = jnp.full_like(mlo_ref[...], -jnp.inf)
        mhi_ref[...] = jnp.full_like(mhi_ref[...], -jnp.inf)

    mlo_ref[...] = jnp.maximum(
        mlo_ref[...], jnp.max(olo_ref[0] + blo[...], axis=0, keepdims=True))
    mhi_ref[...] = jnp.maximum(
        mhi_ref[...], jnp.max(ohi_ref[0] + bhi[...], axis=0, keepdims=True))


def _tc_colmax(o_acc, gc_b, bn):
    n = o_acc.shape[1]
    d = o_acc.shape[2]
    return pl.pallas_call(
        _colmax_body, grid=(n // bn,),
        in_specs=[pl.BlockSpec((1, bn, d), lambda i: (0, i, 0)),
                  pl.BlockSpec((1, bn, d), lambda i: (1, i, 0)),
                  pl.BlockSpec((1, d), lambda i: (0, 0)),
                  pl.BlockSpec((1, d), lambda i: (0, 0))],
        out_specs=[pl.BlockSpec((1, d), lambda i: (0, 0)),
                   pl.BlockSpec((1, d), lambda i: (0, 0))],
        out_shape=[jax.ShapeDtypeStruct((1, d), F32)] * 2,
    )(o_acc, o_acc, gc_b[:d].reshape(1, -1), gc_b[d:].reshape(1, -1))


_MESH = dict(core_axis_name="c", subcore_axis_name="s")


def _fill_const(buf, rows, cols, vec16):
    per_row = cols // 16

    def st(i, carry):
        buf[i // per_row, pl.ds((i % per_row) * 16, 16)] = vec16
        return carry

    lax.fori_loop(0, rows * per_row, st, 0)


def _sc_gather2(a_cat, b_cat, idx_a, idx_b):
    e = idx_a.shape[0]
    d = a_cat.shape[1]
    per_w = e // _NW
    ck = 200
    steps = -(-per_w // ck)
    pairs = (steps + 1) // 2
    last = per_w - ck

    @functools.partial(
        pl.kernel, mesh=plsc.VectorSubcoreMesh(**_MESH),
        out_type=[jax.ShapeDtypeStruct((e, d), jnp.int32)] * 2,
        scratch_types=[pltpu.VMEM((per_w,), jnp.int32),
                       pltpu.VMEM((per_w,), jnp.int32),
                       pltpu.VMEM((ck, d), jnp.int32),
                       pltpu.VMEM((ck, d), jnp.int32),
                       pltpu.VMEM((ck, d), jnp.int32),
                       pltpu.VMEM((ck, d), jnp.int32)]
                      + [pltpu.SemaphoreType.DMA] * 8,
    )
    def k(a_hbm, b_hbm, ia_hbm, ib_hbm, ga_hbm, gb_hbm,
          ia_all, ib_all, bufa0, bufb0, bufa1, bufb1,
          sa0, sb0, sa1, sb1, swa0, swb0, swa1, swb1):
        wid = lax.axis_index("s") * _NC + lax.axis_index("c")
        w0 = wid * per_w
        pltpu.sync_copy(ia_hbm.at[pl.ds(w0, per_w)], ia_all)
        pltpu.sync_copy(ib_hbm.at[pl.ds(w0, per_w)], ib_all)

        def body(j2, carry):
            o0 = jnp.minimum((2 * j2) * ck, last)
            o1 = jnp.minimum((2 * j2 + 1) * ck, last)
            ga0 = pltpu.async_copy(a_hbm.at[ia_all.at[pl.ds(o0, ck)]],
                                   bufa0, sa0)
            gb0 = pltpu.async_copy(b_hbm.at[ib_all.at[pl.ds(o0, ck)]],
                                   bufb0, sb0)
            ga1 = pltpu.async_copy(a_hbm.at[ia_all.at[pl.ds(o1, ck)]],
                                   bufa1, sa1)
            gb1 = pltpu.async_copy(b_hbm.at[ib_all.at[pl.ds(o1, ck)]],
                                   bufb1, sb1)
            ga0.wait()
            gb0.wait()
            wa0 = pltpu.async_copy(bufa0, ga_hbm.at[pl.ds(w0 + o0, ck)], swa0)
            wb0 = pltpu.async_copy(bufb0, gb_hbm.at[pl.ds(w0 + o0, ck)], swb0)
            ga1.wait()
            gb1.wait()
            wa1 = pltpu.async_copy(bufa1, ga_hbm.at[pl.ds(w0 + o1, ck)], swa1)
            wb1 = pltpu.async_copy(bufb1, gb_hbm.at[pl.ds(w0 + o1, ck)], swb1)
            wa0.wait()
            wb0.wait()
            wa1.wait()
            wb1.wait()
            return carry

        lax.fori_loop(0, pairs, body, 0)

    return k(a_cat, b_cat, idx_a, idx_b)


def _sc_scatter_rowsplit(data, idx_cat, n):
    e, d = data.shape
    per_t = e // _NS
    ck = 200
    steps = per_t // ck
    zr = ck
    racc = n + _TR
    n_z = racc // zr
    zsteps = -(-n_z // _NS)
    n_ch = n // zr
    wsteps = -(-n_ch // _NS)

    @functools.partial(
        pl.kernel, mesh=plsc.VectorSubcoreMesh(**_MESH),
        out_type=jax.ShapeDtypeStruct((2, n, d), F32),
        scratch_types=[pltpu.VMEM((ck,), jnp.int32),
                       pltpu.VMEM((ck, d), F32),
                       pltpu.VMEM_SHARED((racc, d), F32)],
    )
    def k(m_hbm, idx_hbm, out_hbm, idx_v, dbuf, acc):
        c = lax.axis_index("c")
        s = lax.axis_index("s")
        _fill_const(dbuf, zr, d, jnp.zeros((16,), F32))

        def zc(i, carry):
            cid = s + i * _NS

            @pl.when(cid < n_z)
            def _():
                pltpu.sync_copy(dbuf, acc.at[pl.ds(cid * zr, zr)])

            return carry

        lax.fori_loop(0, zsteps, zc, 0)
        plsc.subcore_barrier()

        def body(j, carry):
            base = s * per_t + j * ck
            pltpu.sync_copy(idx_hbm.at[pl.ds(c * e + base, ck)], idx_v)
            pltpu.sync_copy(m_hbm.at[pl.ds(base, ck)], dbuf)
            pltpu.sync_copy(dbuf, acc.at[idx_v], add=True)
            return carry

        lax.fori_loop(0, steps, body, 0)
        plsc.subcore_barrier()

        def wb(i, carry):
            cid = s + i * _NS

            @pl.when(cid < n_ch)
            def _():
                r0 = cid * zr
                pltpu.sync_copy(acc.at[pl.ds(r0, zr)], dbuf)

                @pl.when(c == 0)
                def _():
                    pltpu.sync_copy(dbuf, out_hbm.at[0, pl.ds(r0, zr)])

                @pl.when(c == 1)
                def _():
                    pltpu.sync_copy(dbuf, out_hbm.at[1, pl.ds(r0, zr)])

            return carry

        lax.fori_loop(0, wsteps, wb, 0)

    return k(data, idx_cat)


def _sc_counts(idx_cat, e, n):
    d = 128
    per_t = e // _NS
    ck = 200
    steps = per_t // ck
    zr = ck
    racc = n + _TR
    n_z = racc // zr
    zsteps = -(-n_z // _NS)
    n_ch = n // zr
    wsteps = -(-n_ch // _NS)

    @functools.partial(
        pl.kernel, mesh=plsc.VectorSubcoreMesh(**_MESH),
        out_type=jax.ShapeDtypeStruct((2, n, d), F32),
        scratch_types=[pltpu.VMEM((ck,), jnp.int32),
                       pltpu.VMEM((ck, d), F32),
                       pltpu.VMEM_SHARED((racc, d), F32)],
    )
    def k(idx_hbm, out_hbm, idx_v, ones_v, acc):
        c = lax.axis_index("c")
        s = lax.axis_index("s")
        _fill_const(ones_v, zr, d, jnp.zeros((16,), F32))

        def zc(i, carry):
            cid = s + i * _NS

            @pl.when(cid < n_z)
            def _():
                pltpu.sync_copy(ones_v, acc.at[pl.ds(cid * zr, zr)])

            return carry

        lax.fori_loop(0, zsteps, zc, 0)
        plsc.subcore_barrier()
        _fill_const(ones_v, ck, d, jnp.ones((16,), F32))

        def body(j, carry):
            base = c * e + s * per_t + j * ck
            pltpu.sync_copy(idx_hbm.at[pl.ds(base, ck)], idx_v)
            pltpu.sync_copy(ones_v, acc.at[idx_v], add=True)
            return carry

        lax.fori_loop(0, steps, body, 0)
        plsc.subcore_barrier()

        def wb(i, carry):
            cid = s + i * _NS

            @pl.when(cid < n_ch)
            def _():
                r0 = cid * zr
                pltpu.sync_copy(acc.at[pl.ds(r0, zr)], ones_v)

                @pl.when(c == 0)
                def _():
                    pltpu.sync_copy(ones_v, out_hbm.at[0, pl.ds(r0, zr)])

                @pl.when(c == 1)
                def _():
                    pltpu.sync_copy(ones_v, out_hbm.at[1, pl.ds(r0, zr)])

            return carry

        lax.fori_loop(0, wsteps, wb, 0)

    return k(idx_cat)


def _sc_spmm(sup_cat, src2, dst, n):
    e = dst.shape[0]
    d = sup_cat.shape[1]
    per_t = e // _NS
    ck = 200
    steps = per_t // ck
    zr = ck
    n_ch = n // zr
    wsteps = -(-n_ch // _NS)

    @functools.partial(
        pl.kernel, mesh=plsc.VectorSubcoreMesh(**_MESH),
        out_type=jax.ShapeDtypeStruct((2, n, d), F32),
        scratch_types=[pltpu.VMEM((ck,), jnp.int32),
                       pltpu.VMEM((ck,), jnp.int32),
                       pltpu.VMEM((ck, d), F32),
                       pltpu.VMEM_SHARED((n, d), F32),
                       pltpu.SemaphoreType.DMA],
    )
    def k(sup_hbm, src_hbm, dst_hbm, out_hbm, is_v, id_v, gbuf, acc, sem):
        c = lax.axis_index("c")
        s = lax.axis_index("s")
        _fill_const(gbuf, zr, d, jnp.zeros((16,), F32))

        def zc(i, carry):
            cid = s + i * _NS

            @pl.when(cid < n_ch)
            def _():
                pltpu.sync_copy(gbuf, acc.at[pl.ds(cid * zr, zr)])

            return carry

        lax.fori_loop(0, wsteps, zc, 0)
        plsc.subcore_barrier()

        def body(j, carry):
            base = s * per_t + j * ck
            pltpu.sync_copy(src_hbm.at[pl.ds(c * e + base, ck)], is_v)
            pltpu.sync_copy(dst_hbm.at[pl.ds(base, ck)], id_v)
            pltpu.async_copy(sup_hbm.at[is_v], gbuf, sem).wait()
            pltpu.sync_copy(gbuf, acc.at[id_v], add=True)
            return carry

        lax.fori_loop(0, steps, body, 0)
        plsc.subcore_barrier()

        def wb(i, carry):
            cid = s + i * _NS

            @pl.when(cid < n_ch)
            def _():
                r0 = cid * zr
                pltpu.sync_copy(acc.at[pl.ds(r0, zr)], gbuf)

                @pl.when(c == 0)
                def _():
                    pltpu.sync_copy(gbuf, out_hbm.at[0, pl.ds(r0, zr)])

                @pl.when(c == 1)
                def _():
                    pltpu.sync_copy(gbuf, out_hbm.at[1, pl.ds(r0, zr)])

            return carry

        lax.fori_loop(0, wsteps, wb, 0)

    return k(sup_cat, src2, dst)


def kernel(x, l_e, edge_index, edge_label, mlp_v, mlp_e, mlp_edge1,
           mlp_edge0, mlp_aggr, gc_w, gc_b):
    n, emb = x.shape
    e = l_e.shape[0]
    bn = 2000
    eb = 1000
    src = edge_index[0].astype(jnp.int32)
    dst = edge_index[1].astype(jnp.int32)
    mask_f = (edge_label == 1).astype(F32)[:, None]
    off = jnp.where(edge_label == 1, 0, n).astype(jnp.int32)
    src_adj = src + off
    dst_adj = dst + off
    trash = (n + dst % _TR).astype(jnp.int32)
    idx_mail = jnp.concatenate([
        jnp.where(edge_label == 1, dst, trash),
        jnp.where(edge_label == 0, dst, trash)]).astype(jnp.int32)
    src2 = jnp.concatenate([src, src + n]).astype(jnp.int32)

    h = _tc_mlp3(x, mlp_v, bn)
    c_sel = _tc_edge_embed(l_e, mask_f, mlp_e, mlp_edge1, mlp_edge0, eb)
    counts = _sc_counts(idx_mail, e, n)

    wa1 = mlp_edge1[0][:emb]
    wa0 = mlp_edge0[0][:emb]
    wb1 = mlp_edge1[0][emb:2 * emb]
    wb0 = mlp_edge0[0][emb:2 * emb]
    w4 = [jnp.stack([wa1[:, 0::2], wa0[:, 0::2]]),
          jnp.stack([wa1[:, 1::2], wa0[:, 1::2]]),
          jnp.stack([wb1[:, 0::2], wb0[:, 0::2]]),
          jnp.stack([wb1[:, 1::2], wb0[:, 1::2]])]

    for _ in range(2):
        a_cat, b_cat = _tc_tables(h, w4, bn)
        ga, gb = _sc_gather2(a_cat, b_cat, src_adj, dst_adj)
        m = _tc_edge_mlp(ga, gb, c_sel, mask_f, mlp_edge1, mlp_edge0, eb)
        s_mail = _sc_scatter_rowsplit(m, idx_mail, n)
        h = _tc_aggr(h, s_mail, counts, mlp_aggr, bn)

    sup_cat = _tc_support(h, gc_w, bn)
    o_acc = _sc_spmm(sup_cat, src2, dst, n)
    mlo, mhi = _tc_colmax(o_acc, gc_b, bn)
    return jnp.concatenate([mlo[0], mhi[0]], axis=0)


# speedup vs baseline: 3.2922x; 1.1252x over previous
import functools

import jax
import jax.numpy as jnp
from jax import lax
from jax.experimental import pallas as pl
from jax.experimental.pallas import tpu as pltpu
from jax.experimental.pallas import tpu_sc as plsc

F32 = jnp.float32
_NC, _NS = 2, 16
_NW = _NC * _NS
_TR = 800


def _dot(a, b):
    return jnp.dot(a.astype(jnp.bfloat16), b.astype(jnp.bfloat16),
                   preferred_element_type=F32)


def _pack_bf16(even_f32, odd_f32):
    ae = lax.bitcast_convert_type(
        even_f32.astype(jnp.bfloat16).astype(F32), jnp.int32)
    ao = lax.bitcast_convert_type(
        odd_f32.astype(jnp.bfloat16).astype(F32), jnp.int32)
    return lax.shift_right_logical(ae, 16) | ao


def _unpack_bf16(xi):
    lo = lax.bitcast_convert_type(lax.shift_left(xi, 16), F32)
    hi = lax.bitcast_convert_type(xi & jnp.int32(-65536), F32)
    return lo, hi


def _mlp3_body(x_ref, w1, b1, w2, b2, w3, b3, o_ref):
    h = jnp.maximum(_dot(x_ref[...], w1[...]) + b1[...], 0.0)
    h = jnp.maximum(_dot(h, w2[...]) + b2[...], 0.0)
    o_ref[...] = _dot(h, w3[...]) + b3[...]


def _tc_mlp3(x, p, bn):
    n, din = x.shape
    dout = p[4].shape[1]
    args = [x, p[0], p[1].reshape(1, -1), p[2], p[3].reshape(1, -1),
            p[4], p[5].reshape(1, -1)]
    in_specs = [pl.BlockSpec((bn, din), lambda i: (i, 0))]
    for a in args[1:]:
        in_specs.append(pl.BlockSpec(a.shape, lambda i: (0, 0)))
    return pl.pallas_call(
        _mlp3_body, grid=(n // bn,), in_specs=in_specs,
        out_specs=pl.BlockSpec((bn, dout), lambda i: (i, 0)),
        out_shape=jax.ShapeDtypeStruct((n, dout), F32),
    )(*args)


def _edge_embed_body(le_ref, mk_ref, w1, b1, w2, b2, w3, b3,
                     wc1e, wc1o, bc1e, bc1o, wc0e, wc0o, bc0e, bc0o, c_ref):
    h = jnp.maximum(le_ref[...] * w1[...] + b1[...], 0.0)
    h = jnp.maximum(_dot(h, w2[...]) + b2[...], 0.0)
    h = _dot(h, w3[...]) + b3[...]
    mk = mk_ref[...].astype(F32)
    c1e = _dot(h, wc1e[...]) + bc1e[...]
    c0e = _dot(h, wc0e[...]) + bc0e[...]
    c1o = _dot(h, wc1o[...]) + bc1o[...]
    c0o = _dot(h, wc0o[...]) + bc0o[...]
    ce = c0e + mk * (c1e - c0e)
    co = c0o + mk * (c1o - c0o)
    c_ref[...] = _pack_bf16(ce, co)


def _tc_edge_embed(l_e, mask_f, mlp_e, mlp_edge1, mlp_edge0, eb):
    e = l_e.shape[0]
    emb = mlp_e[4].shape[1]
    wc1 = mlp_edge1[0][2 * emb:]
    wc0 = mlp_edge0[0][2 * emb:]
    args = [l_e, mask_f,
            mlp_e[0], mlp_e[1].reshape(1, -1),
            mlp_e[2], mlp_e[3].reshape(1, -1),
            mlp_e[4], mlp_e[5].reshape(1, -1),
            wc1[:, 0::2], wc1[:, 1::2],
            mlp_edge1[1][0::2].reshape(1, -1), mlp_edge1[1][1::2].reshape(1, -1),
            wc0[:, 0::2], wc0[:, 1::2],
            mlp_edge0[1][0::2].reshape(1, -1), mlp_edge0[1][1::2].reshape(1, -1)]
    h1 = mlp_edge1[0].shape[1]
    in_specs = [pl.BlockSpec((eb, 1), lambda i: (i, 0)),
                pl.BlockSpec((eb, 1), lambda i: (i, 0))]
    for a in args[2:]:
        in_specs.append(pl.BlockSpec(a.shape, lambda i: (0, 0)))
    return pl.pallas_call(
        _edge_embed_body, grid=(e // eb,), in_specs=in_specs,
        out_specs=pl.BlockSpec((eb, h1 // 2), lambda i: (i, 0)),
        out_shape=jax.ShapeDtypeStruct((e, h1 // 2), jnp.int32),
    )(*args)


def _tables_body(h_ref, wae, wao, wbe, wbo, a_ref, b_ref):
    hh = h_ref[...]
    a_ref[...] = _pack_bf16(_dot(hh, wae[0]), _dot(hh, wao[0]))
    b_ref[...] = _pack_bf16(_dot(hh, wbe[0]), _dot(hh, wbo[0]))


def _tc_tables(h, w4, bn):
    n, k = h.shape
    m = w4[0].shape[2]
    nb = n // bn
    in_specs = [pl.BlockSpec((bn, k), lambda b, i: (i, 0))]
    for _ in range(4):
        in_specs.append(pl.BlockSpec((1, k, m), lambda b, i: (b, 0, 0)))
    return pl.pallas_call(
        _tables_body, grid=(2, nb), in_specs=in_specs,
        out_specs=[pl.BlockSpec((bn, m), lambda b, i: (b * nb + i, 0)),
                   pl.BlockSpec((bn, m), lambda b, i: (b * nb + i, 0))],
        out_shape=[jax.ShapeDtypeStruct((2 * n, m), jnp.int32)] * 2,
    )(h, *w4)


def _edge_mlp_body(ga_ref, gb_ref, c_ref, mk_ref,
                   w21e, w21o, b21, w31, b31,
                   w20e, w20o, b20, w30, b30, m_ref):
    gae, gao = _unpack_bf16(ga_ref[...])
    gbe, gbo = _unpack_bf16(gb_ref[...])
    ce, co = _unpack_bf16(c_ref[...])
    ze = jnp.maximum(gae + gbe + ce, 0.0)
    zo = jnp.maximum(gao + gbo + co, 0.0)
    u1 = jnp.maximum(_dot(ze, w21e[...]) + _dot(zo, w21o[...]) + b21[...], 0.0)
    m1 = _dot(u1, w31[...]) + b31[...]
    u0 = jnp.maximum(_dot(ze, w20e[...]) + _dot(zo, w20o[...]) + b20[...], 0.0)
    m0 = _dot(u0, w30[...]) + b30[...]
    m_ref[...] = m0 + mk_ref[...].astype(F32) * (m1 - m0)


def _tc_edge_mlp(ga, gb, c_sel, mask_f, mlp_edge1, mlp_edge0, eb):
    e, hp = ga.shape
    emb = mlp_edge1[4].shape[1]
    args = [ga, gb, c_sel, mask_f,
            mlp_edge1[2][0::2], mlp_edge1[2][1::2],
            mlp_edge1[3].reshape(1, -1),
            mlp_edge1[4], mlp_edge1[5].reshape(1, -1),
            mlp_edge0[2][0::2], mlp_edge0[2][1::2],
            mlp_edge0[3].reshape(1, -1),
            mlp_edge0[4], mlp_edge0[5].reshape(1, -1)]
    in_specs = [pl.BlockSpec((eb, hp), lambda i: (i, 0)),
                pl.BlockSpec((eb, hp), lambda i: (i, 0)),
                pl.BlockSpec((eb, hp), lambda i: (i, 0)),
                pl.BlockSpec((eb, 1), lambda i: (i, 0))]
    for a in args[4:]:
        in_specs.append(pl.BlockSpec(a.shape, lambda i: (0, 0)))
    return pl.pallas_call(
        _edge_mlp_body, grid=(e // eb,), in_specs=in_specs,
        out_specs=pl.BlockSpec((eb, emb), lambda i: (i, 0)),
        out_shape=jax.ShapeDtypeStruct((e, emb), F32),
    )(*args)


def _aggr_body(h_ref, s1_ref, s0_ref, c1_ref, c0_ref,
               wh, w1, w0, b1, w2, b2, w3, b3, o_ref):
    hh = h_ref[...]
    sc1 = 1.0 / jnp.maximum(c1_ref[0][:, :1], 1.0)
    sc0 = 1.0 / jnp.maximum(c0_ref[0][:, :1], 1.0)
    u = _dot(hh, wh[...]) + b1[...]
    u += _dot(s1_ref[0] * sc1, w1[...])
    u += _dot(s0_ref[0] * sc0, w0[...])
    u = jnp.maximum(u, 0.0)
    u = jnp.maximum(_dot(u, w2[...]) + b2[...], 0.0)
    o_ref[...] = _dot(u, w3[...]) + b3[...] + hh


def _tc_aggr(h, s_mail, counts, mlp_aggr, bn):
    n, emb = h.shape
    nb = n // bn
    w = mlp_aggr[0]
    args = [h, s_mail, s_mail, counts, counts,
            w[:emb], w[emb:2 * emb], w[2 * emb:],
            mlp_aggr[1].reshape(1, -1),
            mlp_aggr[2], mlp_aggr[3].reshape(1, -1),
            mlp_aggr[4], mlp_aggr[5].reshape(1, -1)]
    in_specs = [pl.BlockSpec((bn, emb), lambda i: (i, 0)),
                pl.BlockSpec((1, bn, emb), lambda i: (0, i, 0)),
                pl.BlockSpec((1, bn, emb), lambda i: (1, i, 0)),
                pl.BlockSpec((1, bn, emb), lambda i: (0, i, 0)),
                pl.BlockSpec((1, bn, emb), lambda i: (1, i, 0))]
    for a in args[5:]:
        in_specs.append(pl.BlockSpec(a.shape, lambda i: (0, 0)))
    return pl.pallas_call(
        _aggr_body, grid=(nb,), in_specs=in_specs,
        out_specs=pl.BlockSpec((bn, emb), lambda i: (i, 0)),
        out_shape=jax.ShapeDtypeStruct((n, emb), F32),
    )(*args)


def _support_body(h_ref, w_ref, o_ref):
    o_ref[...] = _dot(h_ref[...], w_ref[0])


def _tc_support(h, gc_w, bn):
    n, emb = h.shape
    go = gc_w.shape[1]
    d = go // 2
    nb = n // bn
    w_s = jnp.stack([gc_w[:, :d], gc_w[:, d:]])
    return pl.pallas_call(
        _support_body, grid=(2, nb),
        in_specs=[pl.BlockSpec((bn, emb), lambda b, i: (i, 0)),
                  pl.BlockSpec((1, emb, d), lambda b, i: (b, 0, 0))],
        out_specs=pl.BlockSpec((bn, d), lambda b, i: (b * nb + i, 0)),
        out_shape=jax.ShapeDtypeStruct((2 * n, d), F32),
    )(h, w_s)


def _colmax_body(olo_ref, ohi_ref, blo, bhi, mlo_ref, mhi_ref):
    i = pl.program_id(0)

    @pl.when(i == 0)
    def _():
        mlo_ref[...] = jnp.full_like(mlo_ref[...], -jnp.inf)
        mhi_ref[...] = jnp.full_like(mhi_ref[...], -jnp.inf)

    mlo_ref[...] = jnp.maximum(
        mlo_ref[...], jnp.max(olo_ref[0] + blo[...], axis=0, keepdims=True))
    mhi_ref[...] = jnp.maximum(
        mhi_ref[...], jnp.max(ohi_ref[0] + bhi[...], axis=0, keepdims=True))


def _tc_colmax(o_acc, gc_b, bn):
    n = o_acc.shape[1]
    d = o_acc.shape[2]
    return pl.pallas_call(
        _colmax_body, grid=(n // bn,),
        in_specs=[pl.BlockSpec((1, bn, d), lambda i: (0, i, 0)),
                  pl.BlockSpec((1, bn, d), lambda i: (1, i, 0)),
                  pl.BlockSpec((1, d), lambda i: (0, 0)),
                  pl.BlockSpec((1, d), lambda i: (0, 0))],
        out_specs=[pl.BlockSpec((1, d), lambda i: (0, 0)),
                   pl.BlockSpec((1, d), lambda i: (0, 0))],
        out_shape=[jax.ShapeDtypeStruct((1, d), F32)] * 2,
    )(o_acc, o_acc, gc_b[:d].reshape(1, -1), gc_b[d:].reshape(1, -1))


_MESH = dict(core_axis_name="c", subcore_axis_name="s")


def _fill_const(buf, rows, cols, vec16):
    per_row = cols // 16

    def st(i, carry):
        buf[i // per_row, pl.ds((i % per_row) * 16, 16)] = vec16
        return carry

    lax.fori_loop(0, rows * per_row, st, 0)


def _sc_gather2(a_cat, b_cat, idx_a, idx_b):
    e = idx_a.shape[0]
    d = a_cat.shape[1]
    per_w = e // _NW
    ck = 200
    steps = -(-per_w // ck)
    pairs = (steps + 1) // 2
    last = per_w - ck

    @functools.partial(
        pl.kernel, mesh=plsc.VectorSubcoreMesh(**_MESH),
        out_type=[jax.ShapeDtypeStruct((e, d), jnp.int32)] * 2,
        scratch_types=[pltpu.VMEM((per_w,), jnp.int32),
                       pltpu.VMEM((per_w,), jnp.int32),
                       pltpu.VMEM((ck, d), jnp.int32),
                       pltpu.VMEM((ck, d), jnp.int32),
                       pltpu.VMEM((ck, d), jnp.int32),
                       pltpu.VMEM((ck, d), jnp.int32)]
                      + [pltpu.SemaphoreType.DMA] * 8,
    )
    def k(a_hbm, b_hbm, ia_hbm, ib_hbm, ga_hbm, gb_hbm,
          ia_all, ib_all, bufa0, bufb0, bufa1, bufb1,
          sa0, sb0, sa1, sb1, swa0, swb0, swa1, swb1):
        wid = lax.axis_index("s") * _NC + lax.axis_index("c")
        w0 = wid * per_w
        pltpu.sync_copy(ia_hbm.at[pl.ds(w0, per_w)], ia_all)
        pltpu.sync_copy(ib_hbm.at[pl.ds(w0, per_w)], ib_all)

        def body(j2, carry):
            o0 = jnp.minimum((2 * j2) * ck, last)
            o1 = jnp.minimum((2 * j2 + 1) * ck, last)
            ga0 = pltpu.async_copy(a_hbm.at[ia_all.at[pl.ds(o0, ck)]],
                                   bufa0, sa0)
            gb0 = pltpu.async_copy(b_hbm.at[ib_all.at[pl.ds(o0, ck)]],
                                   bufb0, sb0)
            ga1 = pltpu.async_copy(a_hbm.at[ia_all.at[pl.ds(o1, ck)]],
                                   bufa1, sa1)
            gb1 = pltpu.async_copy(b_hbm.at[ib_all.at[pl.ds(o1, ck)]],
                                   bufb1, sb1)
            ga0.wait()
            gb0.wait()
            wa0 = pltpu.async_copy(bufa0, ga_hbm.at[pl.ds(w0 + o0, ck)], swa0)
            wb0 = pltpu.async_copy(bufb0, gb_hbm.at[pl.ds(w0 + o0, ck)], swb0)
            ga1.wait()
            gb1.wait()
            wa1 = pltpu.async_copy(bufa1, ga_hbm.at[pl.ds(w0 + o1, ck)], swa1)
            wb1 = pltpu.async_copy(bufb1, gb_hbm.at[pl.ds(w0 + o1, ck)], swb1)
            wa0.wait()
            wb0.wait()
            wa1.wait()
            wb1.wait()
            return carry

        lax.fori_loop(0, pairs, body, 0)

    return k(a_cat, b_cat, idx_a, idx_b)


def _sc_scatter_rowsplit(data, idx_cat, n):
    e, d = data.shape
    per_t = e // _NS
    ck = 200
    steps = per_t // ck
    zr = ck
    racc = n + _TR
    n_z = racc // zr
    zsteps = -(-n_z // _NS)
    n_ch = n // zr
    wsteps = -(-n_ch // _NS)

    @functools.partial(
        pl.kernel, mesh=plsc.VectorSubcoreMesh(**_MESH),
        out_type=jax.ShapeDtypeStruct((2, n, d), F32),
        scratch_types=[pltpu.VMEM((ck,), jnp.int32),
                       pltpu.VMEM((ck, d), F32),
                       pltpu.VMEM_SHARED((racc, d), F32)],
    )
    def k(m_hbm, idx_hbm, out_hbm, idx_v, dbuf, acc):
        c = lax.axis_index("c")
        s = lax.axis_index("s")
        _fill_const(dbuf, zr, d, jnp.zeros((16,), F32))

        def zc(i, carry):
            cid = s + i * _NS

            @pl.when(cid < n_z)
            def _():
                pltpu.sync_copy(dbuf, acc.at[pl.ds(cid * zr, zr)])

            return carry

        lax.fori_loop(0, zsteps, zc, 0)
        plsc.subcore_barrier()

        def body(j, carry):
            base = s * per_t + j * ck
            pltpu.sync_copy(idx_hbm.at[pl.ds(c * e + base, ck)], idx_v)
            pltpu.sync_copy(m_hbm.at[pl.ds(base, ck)], dbuf)
            pltpu.sync_copy(dbuf, acc.at[idx_v], add=True)
            return carry

        lax.fori_loop(0, steps, body, 0)
        plsc.subcore_barrier()

        def wb(i, carry):
            cid = s + i * _NS

            @pl.when(cid < n_ch)
            def _():
                r0 = cid * zr
                pltpu.sync_copy(acc.at[pl.ds(r0, zr)], dbuf)

                @pl.when(c == 0)
                def _():
                    pltpu.sync_copy(dbuf, out_hbm.at[0, pl.ds(r0, zr)])

                @pl.when(c == 1)
                def _():
                    pltpu.sync_copy(dbuf, out_hbm.at[1, pl.ds(r0, zr)])

            return carry

        lax.fori_loop(0, wsteps, wb, 0)

    return k(data, idx_cat)


def _sc_counts(idx_cat, e, n):
    d = 128
    per_t = e // _NS
    ck = 200
    steps = per_t // ck
    zr = ck
    racc = n + _TR
    n_z = racc // zr
    zsteps = -(-n_z // _NS)
    n_ch = n // zr
    wsteps = -(-n_ch // _NS)

    @functools.partial(
        pl.kernel, mesh=plsc.VectorSubcoreMesh(**_MESH),
        out_type=jax.ShapeDtypeStruct((2, n, d), F32),
        scratch_types=[pltpu.VMEM((ck,), jnp.int32),
                       pltpu.VMEM((ck, d), F32),
                       pltpu.VMEM_SHARED((racc, d), F32)],
    )
    def k(idx_hbm, out_hbm, idx_v, ones_v, acc):
        c = lax.axis_index("c")
        s = lax.axis_index("s")
        _fill_const(ones_v, zr, d, jnp.zeros((16,), F32))

        def zc(i, carry):
            cid = s + i * _NS

            @pl.when(cid < n_z)
            def _():
                pltpu.sync_copy(ones_v, acc.at[pl.ds(cid * zr, zr)])

            return carry

        lax.fori_loop(0, zsteps, zc, 0)
        plsc.subcore_barrier()
        _fill_const(ones_v, ck, d, jnp.ones((16,), F32))

        def body(j, carry):
            base = c * e + s * per_t + j * ck
            pltpu.sync_copy(idx_hbm.at[pl.ds(base, ck)], idx_v)
            pltpu.sync_copy(ones_v, acc.at[idx_v], add=True)
            return carry

        lax.fori_loop(0, steps, body, 0)
        plsc.subcore_barrier()

        def wb(i, carry):
            cid = s + i * _NS

            @pl.when(cid < n_ch)
            def _():
                r0 = cid * zr
                pltpu.sync_copy(acc.at[pl.ds(r0, zr)], ones_v)

                @pl.when(c == 0)
                def _():
                    pltpu.sync_copy(ones_v, out_hbm.at[0, pl.ds(r0, zr)])

                @pl.when(c == 1)
                def _():
                    pltpu.sync_copy(ones_v, out_hbm.at[1, pl.ds(r0, zr)])

            return carry

        lax.fori_loop(0, wsteps, wb, 0)

    return k(idx_cat)


def _sc_spmm(sup_cat, src2, dst, n):
    e = dst.shape[0]
    d = sup_cat.shape[1]
    per_t = e // _NS
    ck = 200
    steps = per_t // ck
    zr = ck
    n_ch = n // zr
    wsteps = -(-n_ch // _NS)

    @functools.partial(
        pl.kernel, mesh=plsc.VectorSubcoreMesh(**_MESH),
        out_type=jax.ShapeDtypeStruct((2, n, d), F32),
        scratch_types=[pltpu.VMEM((ck,), jnp.int32),
                       pltpu.VMEM((ck,), jnp.int32),
                       pltpu.VMEM((ck, d), F32),
                       pltpu.VMEM_SHARED((n, d), F32),
                       pltpu.SemaphoreType.DMA],
    )
    def k(sup_hbm, src_hbm, dst_hbm, out_hbm, is_v, id_v, gbuf, acc, sem):
        c = lax.axis_index("c")
        s = lax.axis_index("s")
        _fill_const(gbuf, zr, d, jnp.zeros((16,), F32))

        def zc(i, carry):
            cid = s + i * _NS

            @pl.when(cid < n_ch)
            def _():
                pltpu.sync_copy(gbuf, acc.at[pl.ds(cid * zr, zr)])

            return carry

        lax.fori_loop(0, wsteps, zc, 0)
        plsc.subcore_barrier()

        def body(j, carry):
            base = s * per_t + j * ck
            pltpu.sync_copy(src_hbm.at[pl.ds(c * e + base, ck)], is_v)
            pltpu.sync_copy(dst_hbm.at[pl.ds(base, ck)], id_v)
            pltpu.async_copy(sup_hbm.at[is_v], gbuf, sem).wait()
            pltpu.sync_copy(gbuf, acc.at[id_v], add=True)
            return carry

        lax.fori_loop(0, steps, body, 0)
        plsc.subcore_barrier()

        def wb(i, carry):
            cid = s + i * _NS

            @pl.when(cid < n_ch)
            def _():
                r0 = cid * zr
                pltpu.sync_copy(acc.at[pl.ds(r0, zr)], gbuf)

                @pl.when(c == 0)
                def _():
                    pltpu.sync_copy(gbuf, out_hbm.at[0, pl.ds(r0, zr)])

                @pl.when(c == 1)
                def _():
                    pltpu.sync_copy(gbuf, out_hbm.at[1, pl.ds(r0, zr)])

            return carry

        lax.fori_loop(0, wsteps, wb, 0)

    return k(sup_cat, src2, dst)


def kernel(x, l_e, edge_index, edge_label, mlp_v, mlp_e, mlp_edge1,
           mlp_edge0, mlp_aggr, gc_w, gc_b):
    n, emb = x.shape
    e = l_e.shape[0]
    bn = 2000
    eb = 2000
    src = edge_index[0].astype(jnp.int32)
    dst = edge_index[1].astype(jnp.int32)
    mask_f = (edge_label == 1).astype(jnp.bfloat16)[:, None]
    off = jnp.where(edge_label == 1, 0, n).astype(jnp.int32)
    src_adj = src + off
    dst_adj = dst + off
    trash = (n + dst % _TR).astype(jnp.int32)
    idx_mail = jnp.concatenate([
        jnp.where(edge_label == 1, dst, trash),
        jnp.where(edge_label == 0, dst, trash)]).astype(jnp.int32)
    src2 = jnp.concatenate([src, src + n]).astype(jnp.int32)

    h = _tc_mlp3(x, mlp_v, bn)
    c_sel = _tc_edge_embed(l_e, mask_f, mlp_e, mlp_edge1, mlp_edge0, eb)
    counts = _sc_counts(idx_mail, e, n)

    wa1 = mlp_edge1[0][:emb]
    wa0 = mlp_edge0[0][:emb]
    wb1 = mlp_edge1[0][emb:2 * emb]
    wb0 = mlp_edge0[0][emb:2 * emb]
    w4 = [jnp.stack([wa1[:, 0::2], wa0[:, 0::2]]),
          jnp.stack([wa1[:, 1::2], wa0[:, 1::2]]),
          jnp.stack([wb1[:, 0::2], wb0[:, 0::2]]),
          jnp.stack([wb1[:, 1::2], wb0[:, 1::2]])]

    for _ in range(2):
        a_cat, b_cat = _tc_tables(h, w4, bn)
        ga, gb = _sc_gather2(a_cat, b_cat, src_adj, dst_adj)
        m = _tc_edge_mlp(ga, gb, c_sel, mask_f, mlp_edge1, mlp_edge0, eb)
        s_mail = _sc_scatter_rowsplit(m, idx_mail, n)
        h = _tc_aggr(h, s_mail, counts, mlp_aggr, bn)

    sup_cat = _tc_support(h, gc_w, bn)
    o_acc = _sc_spmm(sup_cat, src2, dst, n)
    mlo, mhi = _tc_colmax(o_acc, gc_b, bn)
    return jnp.concatenate([mlo[0], mhi[0]], axis=0)
